# Initial kernel scaffold; baseline (speedup 1.0000x reference)
import jax
import jax.numpy as jnp
from jax.experimental import pallas as pl


def kernel(x, edge_index, batch, center_node_index, mask, graph_id_index, W0, b0, initial_embed, Wg, a_src, a_dst, W1, b1, W2, b2):
    raise NotImplementedError("write your pallas kernel here")


# speedup vs baseline: 25.0496x; 25.0496x over previous
import functools

import jax
import jax.numpy as jnp
from jax import lax
from jax.experimental import pallas as pl
from jax.experimental.pallas import tpu as pltpu
from jax.experimental.pallas import tpu_sc as plsc

N = 10000
E = 320000
F = 128
H = 128
HEADS = 3
G = 100
CITIES = N // G

MSGW = 192
CH1 = 80
EPT1 = E // 32
NCH1 = EPT1 // CH1
CH2 = 32
EPT2 = E // 16
NCH2 = EPT2 // CH2
ROWS_PER_TILE = N // 16
NB = 1000
NBLK = N // NB


def _w_body(W0_r, b0_r, ie_r, Wg_r, asrc_r, adst_r, Wf_r, V_r, cc_r, csum_r):
    csum = jnp.zeros((1, 2 * H), jnp.float32)
    vcols = []
    dcols = []
    ccs = []
    ccd = []
    for h in range(HEADS):
        Wg_top = Wg_r[h, :H, :]
        Wg_bot = Wg_r[h, H:, :]
        Wf = jnp.einsum('kf,ko->fo', W0_r[...], Wg_bot,
                        preferred_element_type=jnp.float32)
        Wf_r[h, :, :] = Wf
        const = (jnp.einsum('xk,ko->xo', ie_r[...], Wg_top,
                            preferred_element_type=jnp.float32)
                 + jnp.einsum('xk,ko->xo', b0_r[...], Wg_bot,
                              preferred_element_type=jnp.float32))
        csum = csum + const
        asr = asrc_r[h:h + 1, :]
        adr = adst_r[h:h + 1, :]
        vcols.append(jnp.einsum('fo,xo->fx', Wf, asr,
                                preferred_element_type=jnp.float32))
        dcols.append(jnp.einsum('fo,xo->fx', Wf, adr,
                                preferred_element_type=jnp.float32))
        ccs.append(jnp.einsum('xo,yo->xy', const, asr,
                              preferred_element_type=jnp.float32))
        ccd.append(jnp.einsum('xo,yo->xy', const, adr,
                              preferred_element_type=jnp.float32))
    zc = jnp.zeros((F, 2), jnp.float32)
    V_r[...] = jnp.concatenate(vcols + dcols + [zc], axis=1)
    cc_r[...] = jnp.concatenate(ccs + ccd + [jnp.zeros((1, 2), jnp.float32)],
                                axis=1)
    csum_r[...] = csum


def _prep_weights(W0, b0, initial_embed, Wg, a_src, a_dst):
    return pl.pallas_call(
        _w_body,
        out_shape=(
            jax.ShapeDtypeStruct((HEADS, F, 2 * H), jnp.float32),
            jax.ShapeDtypeStruct((F, 8), jnp.float32),
            jax.ShapeDtypeStruct((1, 8), jnp.float32),
            jax.ShapeDtypeStruct((1, 2 * H), jnp.float32),
        ),
    )(W0, b0.reshape(1, H), initial_embed.reshape(1, H), Wg, a_src, a_dst)


def _node_body(x_r, V_r, cc_r, T_r, xs_r, pmax_r):
    xv = x_r[...]
    T = jnp.dot(xv, V_r[...], preferred_element_type=jnp.float32) + cc_r[...]
    T_r[...] = T
    xs_r[0, :, :] = xv[:, :64]
    xs_r[1, :, :] = xv[:, 64:]
    pmax_r[0, :, :] = jnp.max(T, axis=0, keepdims=True)


def _prep_nodes(x, V, cc):
    return pl.pallas_call(
        _node_body,
        grid=(NBLK,),
        in_specs=[
            pl.BlockSpec((NB, F), lambda i: (i, 0)),
            pl.BlockSpec((F, 8), lambda i: (0, 0)),
            pl.BlockSpec((1, 8), lambda i: (0, 0)),
        ],
        out_specs=(
            pl.BlockSpec((NB, 8), lambda i: (i, 0)),
            pl.BlockSpec((2, NB, 64), lambda i: (0, i, 0)),
            pl.BlockSpec((1, 1, 8), lambda i: (i, 0, 0)),
        ),
        out_shape=(
            jax.ShapeDtypeStruct((N, 8), jnp.float32),
            jax.ShapeDtypeStruct((2, N, 64), jnp.float32),
            jax.ShapeDtypeStruct((NBLK, 1, 8), jnp.float32),
        ),
    )(x, V, cc)


def _bound_body(pmax_r, M_r):
    cm = jnp.max(pmax_r[...], axis=0, keepdims=True)
    r = lax.broadcasted_iota(jnp.int32, (8, 16), 0)
    c = lax.broadcasted_iota(jnp.int32, (8, 16), 1)
    sel = jnp.where((c < 3) & ((r == c) | (r == c + 3)), 1.0, 0.0)
    m = jnp.dot(cm, sel, preferred_element_type=jnp.float32)
    m = jnp.where(m > 0, m, 0.2 * m)
    M_r[...] = m


def _reduce_bound(pmax):
    return pl.pallas_call(
        _bound_body,
        out_shape=jax.ShapeDtypeStruct((1, 16), jnp.float32),
    )(pmax)


_SC_PARAMS = pltpu.CompilerParams(use_tc_tiling_on_sc=False,
                                  needs_layout_passes=False)
_LANE = lambda: lax.broadcasted_iota(jnp.int32, (16,), 0)


def _edge_w(tsrc_v, tdst_v, g, m0, m1, m2):
    rows = jnp.full((16,), g * 16, jnp.int32) + _LANE()
    c0 = jnp.full((16,), 0, jnp.int32)
    ws = []
    for h, m in ((0, m0), (1, m1), (2, m2)):
        es = plsc.load_gather(tsrc_v, [rows, c0 + h])
        ed = plsc.load_gather(tdst_v, [rows, c0 + (3 + h)])
        e = es + ed
        e = jnp.where(e < 0, 0.2 * e, e)
        ws.append(jnp.exp(e - jnp.full((16,), m, jnp.float32)))
    return ws


def _den_body(ei_hbm, T_hbm, M_hbm, z_hbm,
              out_hbm,
              src_v, dst_v, tsrc_v, tdst_v, wstag_v, M_v,
              acc_sh, sem):
    cid = lax.axis_index("c")
    sid = lax.axis_index("s")
    pltpu.sync_copy(M_hbm, M_v)
    pltpu.sync_copy(z_hbm,
                    acc_sh.at[pl.ds(sid * ROWS_PER_TILE, ROWS_PER_TILE)])
    zero16 = jnp.zeros((16,), jnp.float32)
    for r in range(CH1):
        wstag_v[r, pl.ds(0, 16)] = zero16
    plsc.subcore_barrier()
    mv = M_v[0, :]
    m0, m1, m2 = mv[0], mv[1], mv[2]

    def chunk(k):
        base = (cid * 16 + sid) * EPT1 + k * CH1
        pltpu.sync_copy(ei_hbm.at[0, pl.ds(base, CH1)], src_v)
        pltpu.sync_copy(ei_hbm.at[1, pl.ds(base, CH1)], dst_v)
        cps = pltpu.async_copy(T_hbm.at[src_v], tsrc_v, sem)
        cpd = pltpu.async_copy(T_hbm.at[dst_v], tdst_v, sem)
        cps.wait()
        cpd.wait()
        for g in range(CH1 // 16):
            w0, w1, w2 = _edge_w(tsrc_v, tdst_v, g, m0, m1, m2)
            rows = jnp.full((16,), g * 16, jnp.int32) + _LANE()
            c0 = jnp.full((16,), 0, jnp.int32)
            plsc.store_scatter(wstag_v, [rows, c0], w0)
            plsc.store_scatter(wstag_v, [rows, c0 + 1], w1)
            plsc.store_scatter(wstag_v, [rows, c0 + 2], w2)
        pltpu.sync_copy(wstag_v, acc_sh.at[dst_v], add=True)

    pl.loop(0, NCH1)(chunk)

    plsc.subcore_barrier()
    pltpu.sync_copy(
        acc_sh.at[pl.ds(sid * ROWS_PER_TILE, ROWS_PER_TILE)],
        out_hbm.at[cid, pl.ds(sid * ROWS_PER_TILE, ROWS_PER_TILE)])


def _sc_denominators(edge_index, T, M, zrows16):
    mesh = plsc.VectorSubcoreMesh(core_axis_name="c", subcore_axis_name="s")
    f = pl.kernel(
        _den_body,
        out_type=jax.ShapeDtypeStruct((2, N, 16), jnp.float32),
        mesh=mesh,
        compiler_params=_SC_PARAMS,
        scratch_types=[
            pltpu.VMEM((CH1,), jnp.int32),
            pltpu.VMEM((CH1,), jnp.int32),
            pltpu.VMEM((CH1, 8), jnp.float32),
            pltpu.VMEM((CH1, 8), jnp.float32),
            pltpu.VMEM((CH1, 16), jnp.float32),
            pltpu.VMEM((1, 16), jnp.float32),
            pltpu.VMEM_SHARED((N, 16), jnp.float32),
            pltpu.SemaphoreType.DMA,
        ],
    )
    return f(edge_index, T, M, zrows16)


def _msg_body(ei_hbm, T_hbm, xf_hbm, M_hbm, z_hbm,
              out_hbm,
              src_v, dst_v, idx_v, tsrc_v, tdst_v, rows_v, stag_v, M_v,
              acc_sh, sem):
    cid = lax.axis_index("c")
    sid = lax.axis_index("s")
    pltpu.sync_copy(M_hbm, M_v)
    pltpu.sync_copy(z_hbm,
                    acc_sh.at[pl.ds(sid * ROWS_PER_TILE, ROWS_PER_TILE)])
    plsc.subcore_barrier()
    mv = M_v[0, :]
    m0, m1, m2 = mv[0], mv[1], mv[2]
    coff = cid * N

    def chunk(k):
        base = sid * EPT2 + k * CH2
        pltpu.sync_copy(ei_hbm.at[0, pl.ds(base, CH2)], src_v)
        pltpu.sync_copy(ei_hbm.at[1, pl.ds(base, CH2)], dst_v)
        for g in range(CH2 // 16):
            sv = src_v[pl.ds(g * 16, 16)]
            idx_v[pl.ds(g * 16, 16)] = sv + jnp.full((16,), coff, jnp.int32)
        cpx = pltpu.async_copy(xf_hbm.at[idx_v], rows_v, sem)
        cps = pltpu.async_copy(T_hbm.at[src_v], tsrc_v, sem)
        cpd = pltpu.async_copy(T_hbm.at[dst_v], tdst_v, sem)
        cpx.wait()
        cps.wait()
        cpd.wait()
        for g in range(CH2 // 16):
            w0, w1, w2 = _edge_w(tsrc_v, tdst_v, g, m0, m1, m2)
            for i in range(16):
                eidx = g * 16 + i
                wb0 = jnp.full((16,), w0[i], jnp.float32)
                wb1 = jnp.full((16,), w1[i], jnp.float32)
                wb2 = jnp.full((16,), w2[i], jnp.float32)
                for cb in range(4):
                    xvv = rows_v[eidx, pl.ds(cb * 16, 16)]
                    stag_v[eidx, pl.ds(cb * 16, 16)] = xvv * wb0
                    stag_v[eidx, pl.ds(64 + cb * 16, 16)] = xvv * wb1
                    stag_v[eidx, pl.ds(128 + cb * 16, 16)] = xvv * wb2
        pltpu.sync_copy(stag_v, acc_sh.at[dst_v], add=True)

    pl.loop(0, NCH2)(chunk)

    plsc.subcore_barrier()
    pltpu.sync_copy(
        acc_sh.at[pl.ds(sid * ROWS_PER_TILE, ROWS_PER_TILE)],
        out_hbm.at[cid, pl.ds(sid * ROWS_PER_TILE, ROWS_PER_TILE)])


def _sc_messages(edge_index, T, xflat, M, zrows):
    mesh = plsc.VectorSubcoreMesh(core_axis_name="c", subcore_axis_name="s")
    f = pl.kernel(
        _msg_body,
        out_type=jax.ShapeDtypeStruct((2, N, MSGW), jnp.float32),
        mesh=mesh,
        compiler_params=_SC_PARAMS,
        scratch_types=[
            pltpu.VMEM((CH2,), jnp.int32),
            pltpu.VMEM((CH2,), jnp.int32),
            pltpu.VMEM((CH2,), jnp.int32),
            pltpu.VMEM((CH2, 8), jnp.float32),
            pltpu.VMEM((CH2, 8), jnp.float32),
            pltpu.VMEM((CH2, 64), jnp.float32),
            pltpu.VMEM((CH2, MSGW), jnp.float32),
            pltpu.VMEM((1, 16), jnp.float32),
            pltpu.VMEM_SHARED((N, MSGW), jnp.float32),
            pltpu.SemaphoreType.DMA,
        ],
    )
    return f(edge_index, T, xflat, M, zrows)


def _post_body(a0_r, a1_r, d0_r, d1_r, x_r, mask_r, Wf_r, csum_r, W1_r, b1_r,
               W2_r, b2_r, lg_r):
    den3 = d0_r[...] + d1_r[...]
    has = den3[:, 0:1] > 0
    osum = jnp.where(has, csum_r[...], 0.0)
    for h in range(HEADS):
        den = den3[:, h:h + 1]
        agg = jnp.concatenate(
            [a0_r[:, h * 64:(h + 1) * 64], a1_r[:, h * 64:(h + 1) * 64]],
            axis=1)
        agg = jnp.where(den > 0, agg / jnp.where(den > 0, den, 1.0), 0.0)
        osum = osum + jnp.dot(agg, Wf_r[h, :, :],
                              preferred_element_type=jnp.float32)
    enc = osum * (1.0 / 3.0)
    enc = jnp.where(enc > 0, enc, jnp.exp(jnp.minimum(enc, 0.0)) - 1.0)
    encg = enc.reshape(NB // CITIES, CITIES, 2 * H)
    center = jnp.broadcast_to(encg[:, 0:1, :], encg.shape)
    crep = center.reshape(NB, 2 * H)
    h1 = (jnp.einsum('nc,kc->nk', crep, W1_r[:, :2 * H],
                     preferred_element_type=jnp.float32)
          + jnp.einsum('nc,kc->nk', enc, W1_r[:, 2 * H:],
                       preferred_element_type=jnp.float32)
          + b1_r[...])
    h1 = jnp.maximum(h1, 0.0)
    lg = (jnp.einsum('nf,of->no', x_r[...], W2_r[:, :F],
                     preferred_element_type=jnp.float32)
          + jnp.einsum('nh,oh->no', h1, W2_r[:, F:],
                       preferred_element_type=jnp.float32)
          + b2_r[...])
    lg_r[...] = lg - jnp.where(mask_r[...], 0.0, 1e6)


def _post(a0, a1, d0, d1, x, mask, Wf, csum, W1, b1, W2, b2):
    return pl.pallas_call(
        _post_body,
        grid=(NBLK,),
        in_specs=[
            pl.BlockSpec((NB, MSGW), lambda i: (i, 0)),
            pl.BlockSpec((NB, MSGW), lambda i: (i, 0)),
            pl.BlockSpec((NB, 16), lambda i: (i, 0)),
            pl.BlockSpec((NB, 16), lambda i: (i, 0)),
            pl.BlockSpec((NB, F), lambda i: (i, 0)),
            pl.BlockSpec((NB, 1), lambda i: (i, 0)),
            pl.BlockSpec((HEADS, F, 2 * H), lambda i: (0, 0, 0)),
            pl.BlockSpec((1, 2 * H), lambda i: (0, 0)),
            pl.BlockSpec((H, 4 * H), lambda i: (0, 0)),
            pl.BlockSpec((1, H), lambda i: (0, 0)),
            pl.BlockSpec((1, H + F), lambda i: (0, 0)),
            pl.BlockSpec((1, 1), lambda i: (0, 0)),
        ],
        out_specs=pl.BlockSpec((NB, 1), lambda i: (i, 0)),
        out_shape=jax.ShapeDtypeStruct((N, 1), jnp.float32),
    )(a0, a1, d0, d1, x, mask, Wf, csum, W1, b1.reshape(1, H), W2,
      b2.reshape(1, 1))


def _sample_body(lg_r, gid_r, samp_r, logp_r):
    iota = lax.broadcasted_iota(jnp.int32, (CITIES, 1), 0)
    for g in range(G):
        l = lg_r[pl.ds(g * CITIES, CITIES), :]
        m = jnp.max(l, axis=0, keepdims=True)
        den = jnp.sum(jnp.exp(l - m), axis=0, keepdims=True)
        cand = jnp.where(l >= m, iota, N)
        samp = jnp.min(cand, axis=0, keepdims=True)
        samp_r[pl.ds(g, 1), :] = samp + gid_r[pl.ds(g, 1), :]
        logp_r[pl.ds(g, 1), :] = -jnp.log(den)


def _sample(logits, gid2d):
    return pl.pallas_call(
        _sample_body,
        out_shape=(
            jax.ShapeDtypeStruct((G, 1), jnp.int32),
            jax.ShapeDtypeStruct((G, 1), jnp.float32),
        ),
    )(logits, gid2d)


def kernel(x, edge_index, batch, center_node_index, mask, graph_id_index,
           W0, b0, initial_embed, Wg, a_src, a_dst, W1, b1, W2, b2):
    Wf, V, cc, csum = _prep_weights(W0, b0, initial_embed, Wg, a_src, a_dst)
    T, xsplit, pmax = _prep_nodes(x, V, cc)
    M = _reduce_bound(pmax.reshape(NBLK, 8))
    zden = jnp.zeros((ROWS_PER_TILE, 16), jnp.float32)
    zmsg = jnp.zeros((ROWS_PER_TILE, MSGW), jnp.float32)
    den = _sc_denominators(edge_index, T, M, zden)
    acc = _sc_messages(edge_index, T, xsplit.reshape(2 * N, 64), M, zmsg)
    logits = _post(acc[0], acc[1], den[0], den[1], x, mask, Wf, csum,
                   W1, b1, W2, b2)
    samp, logp = _sample(logits, graph_id_index.reshape(G, 1))
    return samp.reshape(G), logp.reshape(G)


# speedup vs baseline: 43.8697x; 1.7513x over previous
import functools

import jax
import jax.numpy as jnp
from jax import lax
from jax.experimental import pallas as pl
from jax.experimental.pallas import tpu as pltpu
from jax.experimental.pallas import tpu_sc as plsc

N = 10000
E = 320000
F = 128
H = 128
HEADS = 3
G = 100
CITIES = N // G

MSGW = 192
CH1 = 80
EPT1 = E // 32
NCH1 = EPT1 // CH1
CH2 = 32
EPT2 = E // 16
NCH2 = EPT2 // CH2
ROWS_PER_TILE = N // 16
NB = 1000
NBLK = N // NB


def _w_body(W0_r, b0_r, ie_r, Wg_r, asrc_r, adst_r, Wf_r, V_r, cc_r, csum_r):
    csum = jnp.zeros((1, 2 * H), jnp.float32)
    vcols = []
    dcols = []
    ccs = []
    ccd = []
    for h in range(HEADS):
        Wg_top = Wg_r[h, :H, :]
        Wg_bot = Wg_r[h, H:, :]
        Wf = jnp.einsum('kf,ko->fo', W0_r[...], Wg_bot,
                        preferred_element_type=jnp.float32)
        Wf_r[h, :, :] = Wf
        const = (jnp.einsum('xk,ko->xo', ie_r[...], Wg_top,
                            preferred_element_type=jnp.float32)
                 + jnp.einsum('xk,ko->xo', b0_r[...], Wg_bot,
                              preferred_element_type=jnp.float32))
        csum = csum + const
        asr = asrc_r[h:h + 1, :]
        adr = adst_r[h:h + 1, :]
        vcols.append(jnp.einsum('fo,xo->fx', Wf, asr,
                                preferred_element_type=jnp.float32))
        dcols.append(jnp.einsum('fo,xo->fx', Wf, adr,
                                preferred_element_type=jnp.float32))
        ccs.append(jnp.einsum('xo,yo->xy', const, asr,
                              preferred_element_type=jnp.float32))
        ccd.append(jnp.einsum('xo,yo->xy', const, adr,
                              preferred_element_type=jnp.float32))
    zc = jnp.zeros((F, 2), jnp.float32)
    V_r[...] = jnp.concatenate(vcols + dcols + [zc], axis=1)
    cc_r[...] = jnp.concatenate(ccs + ccd + [jnp.zeros((1, 2), jnp.float32)],
                                axis=1)
    csum_r[...] = csum


def _prep_weights(W0, b0, initial_embed, Wg, a_src, a_dst):
    return pl.pallas_call(
        _w_body,
        out_shape=(
            jax.ShapeDtypeStruct((HEADS, F, 2 * H), jnp.float32),
            jax.ShapeDtypeStruct((F, 8), jnp.float32),
            jax.ShapeDtypeStruct((1, 8), jnp.float32),
            jax.ShapeDtypeStruct((1, 2 * H), jnp.float32),
        ),
    )(W0, b0.reshape(1, H), initial_embed.reshape(1, H), Wg, a_src, a_dst)


def _node_body(x_r, V_r, cc_r, T_r, xs_r, pmax_r):
    xv = x_r[...]
    T = jnp.dot(xv, V_r[...], preferred_element_type=jnp.float32) + cc_r[...]
    T_r[...] = T
    xs_r[0, :, :] = xv[:, :64]
    xs_r[1, :, :] = xv[:, 64:]
    pmax_r[0, :, :] = jnp.max(T, axis=0, keepdims=True)


def _prep_nodes(x, V, cc):
    return pl.pallas_call(
        _node_body,
        grid=(NBLK,),
        in_specs=[
            pl.BlockSpec((NB, F), lambda i: (i, 0)),
            pl.BlockSpec((F, 8), lambda i: (0, 0)),
            pl.BlockSpec((1, 8), lambda i: (0, 0)),
        ],
        out_specs=(
            pl.BlockSpec((NB, 8), lambda i: (i, 0)),
            pl.BlockSpec((2, NB, 64), lambda i: (0, i, 0)),
            pl.BlockSpec((1, 1, 8), lambda i: (i, 0, 0)),
        ),
        out_shape=(
            jax.ShapeDtypeStruct((N, 8), jnp.float32),
            jax.ShapeDtypeStruct((2, N, 64), jnp.float32),
            jax.ShapeDtypeStruct((NBLK, 1, 8), jnp.float32),
        ),
    )(x, V, cc)


def _bound_body(pmax_r, M_r):
    cm = jnp.max(pmax_r[...], axis=0, keepdims=True)
    r = lax.broadcasted_iota(jnp.int32, (8, 16), 0)
    c = lax.broadcasted_iota(jnp.int32, (8, 16), 1)
    sel = jnp.where((c < 3) & ((r == c) | (r == c + 3)), 1.0, 0.0)
    m = jnp.dot(cm, sel, preferred_element_type=jnp.float32)
    m = jnp.where(m > 0, m, 0.2 * m)
    M_r[...] = m


def _reduce_bound(pmax):
    return pl.pallas_call(
        _bound_body,
        out_shape=jax.ShapeDtypeStruct((1, 16), jnp.float32),
    )(pmax)


_SC_PARAMS = pltpu.CompilerParams(use_tc_tiling_on_sc=False,
                                  needs_layout_passes=False)
_LANE = lambda: lax.broadcasted_iota(jnp.int32, (16,), 0)


def _edge_w(tsrc_v, tdst_v, g, m0, m1, m2):
    rows = jnp.full((16,), g * 16, jnp.int32) + _LANE()
    c0 = jnp.full((16,), 0, jnp.int32)
    ws = []
    for h, m in ((0, m0), (1, m1), (2, m2)):
        es = plsc.load_gather(tsrc_v, [rows, c0 + h])
        ed = plsc.load_gather(tdst_v, [rows, c0 + (3 + h)])
        e = es + ed
        e = jnp.where(e < 0, 0.2 * e, e)
        ws.append(jnp.exp(e - jnp.full((16,), m, jnp.float32)))
    return ws


def _den_body(ei_hbm, T_hbm, M_hbm, z_hbm,
              out_hbm, w4_hbm,
              src_v, dst_v, tsrc_v, tdst_v, wstag_v, w4stag_v, M_v,
              acc_sh, sem):
    cid = lax.axis_index("c")
    sid = lax.axis_index("s")
    pltpu.sync_copy(M_hbm, M_v)
    pltpu.sync_copy(z_hbm,
                    acc_sh.at[pl.ds(sid * ROWS_PER_TILE, ROWS_PER_TILE)])
    zero16 = jnp.zeros((16,), jnp.float32)
    for r in range(CH1):
        wstag_v[r, pl.ds(0, 16)] = zero16
    plsc.subcore_barrier()
    mv = M_v[0, :]
    m0, m1, m2 = mv[0], mv[1], mv[2]

    def chunk(k):
        base = (cid * 16 + sid) * EPT1 + k * CH1
        pltpu.sync_copy(ei_hbm.at[0, pl.ds(base, CH1)], src_v)
        pltpu.sync_copy(ei_hbm.at[1, pl.ds(base, CH1)], dst_v)
        cps = pltpu.async_copy(T_hbm.at[src_v], tsrc_v, sem)
        cpd = pltpu.async_copy(T_hbm.at[dst_v], tdst_v, sem)
        cps.wait()
        cpd.wait()
        for g in range(CH1 // 16):
            w0, w1, w2 = _edge_w(tsrc_v, tdst_v, g, m0, m1, m2)
            rows = jnp.full((16,), g * 16, jnp.int32) + _LANE()
            c0 = jnp.full((16,), 0, jnp.int32)
            plsc.store_scatter(wstag_v, [rows, c0], w0)
            plsc.store_scatter(wstag_v, [rows, c0 + 1], w1)
            plsc.store_scatter(wstag_v, [rows, c0 + 2], w2)
            plsc.store_scatter(w4stag_v, [rows, c0], w0)
            plsc.store_scatter(w4stag_v, [rows, c0 + 1], w1)
            plsc.store_scatter(w4stag_v, [rows, c0 + 2], w2)
        pltpu.sync_copy(wstag_v, acc_sh.at[dst_v], add=True)
        pltpu.sync_copy(w4stag_v, w4_hbm.at[pl.ds(base, CH1)])

    pl.loop(0, NCH1)(chunk)

    plsc.subcore_barrier()
    pltpu.sync_copy(
        acc_sh.at[pl.ds(sid * ROWS_PER_TILE, ROWS_PER_TILE)],
        out_hbm.at[cid, pl.ds(sid * ROWS_PER_TILE, ROWS_PER_TILE)])


def _sc_denominators(edge_index, T, M, zrows16):
    mesh = plsc.VectorSubcoreMesh(core_axis_name="c", subcore_axis_name="s")
    f = pl.kernel(
        _den_body,
        out_type=(jax.ShapeDtypeStruct((2, N, 16), jnp.float32),
                  jax.ShapeDtypeStruct((E, 4), jnp.float32)),
        mesh=mesh,
        compiler_params=_SC_PARAMS,
        scratch_types=[
            pltpu.VMEM((CH1,), jnp.int32),
            pltpu.VMEM((CH1,), jnp.int32),
            pltpu.VMEM((CH1, 8), jnp.float32),
            pltpu.VMEM((CH1, 8), jnp.float32),
            pltpu.VMEM((CH1, 16), jnp.float32),
            pltpu.VMEM((CH1, 4), jnp.float32),
            pltpu.VMEM((1, 16), jnp.float32),
            pltpu.VMEM_SHARED((N, 16), jnp.float32),
            pltpu.SemaphoreType.DMA,
        ],
    )
    return f(edge_index, T, M, zrows16)


def _msg_body(ei_hbm, xf_hbm, w4_hbm, z_hbm,
              out_hbm,
              src_v, dst_v, w4_v,
              idx0, idx1, dsc0, dsc1, dsp0, dsp1, w4c0, w4c1,
              rows0, rows1, stag_v,
              acc_sh,
              sin, sr0, sr1, ss):
    cid = lax.axis_index("c")
    sid = lax.axis_index("s")
    pltpu.sync_copy(z_hbm,
                    acc_sh.at[pl.ds(sid * ROWS_PER_TILE, ROWS_PER_TILE)])
    plsc.subcore_barrier()
    coff = cid * N

    idxs = (idx0, idx1)
    dscs = (dsc0, dsc1)
    dsps = (dsp0, dsp1)
    w4cs = (w4c0, w4c1)
    rowss = (rows0, rows1)
    srs = (sr0, sr1)

    def issue_in(j):
        cj = jnp.minimum(j, NCH2 - 1)
        base = sid * EPT2 + cj * CH2
        pltpu.async_copy(ei_hbm.at[0, pl.ds(base, CH2)], src_v, sin)
        pltpu.async_copy(ei_hbm.at[1, pl.ds(base, CH2)], dst_v, sin)
        pltpu.async_copy(w4_hbm.at[pl.ds(base * 4, CH2 * 4)], w4_v, sin)

    def wait_in():
        pltpu.make_async_copy(ei_hbm.at[0, pl.ds(0, CH2)], src_v, sin).wait()
        pltpu.make_async_copy(ei_hbm.at[1, pl.ds(0, CH2)], dst_v, sin).wait()
        pltpu.make_async_copy(w4_hbm.at[pl.ds(0, CH2 * 4)], w4_v, sin).wait()

    def extract(b):
        for g in range(CH2 // 16):
            sl = pl.ds(g * 16, 16)
            idxs[b][sl] = src_v[sl] + jnp.full((16,), coff, jnp.int32)
            dscs[b][sl] = dst_v[sl]
        for g in range(CH2 * 4 // 16):
            sl = pl.ds(g * 16, 16)
            w4cs[b][sl] = w4_v[sl]

    def issue_rows(b):
        pltpu.async_copy(xf_hbm.at[idxs[b]], rowss[b], srs[b])

    def wait_rows(b):
        pltpu.make_async_copy(xf_hbm.at[idxs[b]], rowss[b], srs[b]).wait()

    def issue_scat(b):
        for g in range(CH2 // 16):
            sl = pl.ds(g * 16, 16)
            dsps[b][sl] = dscs[b][sl]
        pltpu.async_copy(stag_v, acc_sh.at[dsps[b]], ss, add=True)

    def wait_scat(b):
        pltpu.make_async_copy(stag_v, acc_sh.at[dsps[b]], ss).wait()

    def compute(b):
        w4c = w4cs[b]
        rows = rowss[b]
        for g in range(CH2 // 16):
            r4 = (jnp.full((16,), g * 16, jnp.int32) + _LANE()) * 4
            wv0 = plsc.load_gather(w4c, [r4])
            wv1 = plsc.load_gather(w4c, [r4 + 1])
            wv2 = plsc.load_gather(w4c, [r4 + 2])
            for i in range(16):
                e = g * 16 + i
                wb0 = jnp.full((16,), wv0[i], jnp.float32)
                wb1 = jnp.full((16,), wv1[i], jnp.float32)
                wb2 = jnp.full((16,), wv2[i], jnp.float32)
                for cb in range(4):
                    xvv = rows[e, pl.ds(cb * 16, 16)]
                    stag_v[e, pl.ds(cb * 16, 16)] = xvv * wb0
                    stag_v[e, pl.ds(64 + cb * 16, 16)] = xvv * wb1
                    stag_v[e, pl.ds(128 + cb * 16, 16)] = xvv * wb2

    def body(j, b, first):
        nb = 1 - b
        wait_in()
        extract(nb)
        issue_in(j + 2)
        issue_rows(nb)
        wait_rows(b)
        if first:
            @pl.when(j > 0)
            def _():
                wait_scat(nb)
        else:
            wait_scat(nb)
        compute(b)
        issue_scat(b)

    issue_in(0)
    wait_in()
    extract(0)
    issue_rows(0)
    issue_in(1)

    def two(j):
        body(j, 0, True)
        body(j + 1, 1, False)

    pl.loop(0, NCH2 - 1, step=2)(two)
    body(NCH2 - 1, (NCH2 - 1) % 2, False)

    wait_scat((NCH2 - 1) % 2)
    wait_rows(NCH2 % 2)
    wait_in()

    plsc.subcore_barrier()
    pltpu.sync_copy(
        acc_sh.at[pl.ds(sid * ROWS_PER_TILE, ROWS_PER_TILE)],
        out_hbm.at[cid, pl.ds(sid * ROWS_PER_TILE, ROWS_PER_TILE)])


def _sc_messages(edge_index, xflat, w4flat, zrows):
    mesh = plsc.VectorSubcoreMesh(core_axis_name="c", subcore_axis_name="s")
    f = pl.kernel(
        _msg_body,
        out_type=jax.ShapeDtypeStruct((2, N, MSGW), jnp.float32),
        mesh=mesh,
        compiler_params=_SC_PARAMS,
        scratch_types=[
            pltpu.VMEM((CH2,), jnp.int32),
            pltpu.VMEM((CH2,), jnp.int32),
            pltpu.VMEM((CH2 * 4,), jnp.float32),
            pltpu.VMEM((CH2,), jnp.int32),
            pltpu.VMEM((CH2,), jnp.int32),
            pltpu.VMEM((CH2,), jnp.int32),
            pltpu.VMEM((CH2,), jnp.int32),
            pltpu.VMEM((CH2,), jnp.int32),
            pltpu.VMEM((CH2,), jnp.int32),
            pltpu.VMEM((CH2 * 4,), jnp.float32),
            pltpu.VMEM((CH2 * 4,), jnp.float32),
            pltpu.VMEM((CH2, 64), jnp.float32),
            pltpu.VMEM((CH2, 64), jnp.float32),
            pltpu.VMEM((CH2, MSGW), jnp.float32),
            pltpu.VMEM_SHARED((N, MSGW), jnp.float32),
            pltpu.SemaphoreType.DMA,
            pltpu.SemaphoreType.DMA,
            pltpu.SemaphoreType.DMA,
            pltpu.SemaphoreType.DMA,
        ],
    )
    return f(edge_index, xflat, w4flat, zrows)


def _post_body(a0_r, a1_r, d0_r, d1_r, x_r, mask_r, Wf_r, csum_r, W1_r, b1_r,
               W2_r, b2_r, lg_r):
    den3 = d0_r[...] + d1_r[...]
    has = den3[:, 0:1] > 0
    osum = jnp.where(has, csum_r[...], 0.0)
    for h in range(HEADS):
        den = den3[:, h:h + 1]
        agg = jnp.concatenate(
            [a0_r[:, h * 64:(h + 1) * 64], a1_r[:, h * 64:(h + 1) * 64]],
            axis=1)
        agg = jnp.where(den > 0, agg / jnp.where(den > 0, den, 1.0), 0.0)
        osum = osum + jnp.dot(agg, Wf_r[h, :, :],
                              preferred_element_type=jnp.float32)
    enc = osum * (1.0 / 3.0)
    enc = jnp.where(enc > 0, enc, jnp.exp(jnp.minimum(enc, 0.0)) - 1.0)
    encg = enc.reshape(NB // CITIES, CITIES, 2 * H)
    center = jnp.broadcast_to(encg[:, 0:1, :], encg.shape)
    crep = center.reshape(NB, 2 * H)
    h1 = (jnp.einsum('nc,kc->nk', crep, W1_r[:, :2 * H],
                     preferred_element_type=jnp.float32)
          + jnp.einsum('nc,kc->nk', enc, W1_r[:, 2 * H:],
                       preferred_element_type=jnp.float32)
          + b1_r[...])
    h1 = jnp.maximum(h1, 0.0)
    lg = (jnp.einsum('nf,of->no', x_r[...], W2_r[:, :F],
                     preferred_element_type=jnp.float32)
          + jnp.einsum('nh,oh->no', h1, W2_r[:, F:],
                       preferred_element_type=jnp.float32)
          + b2_r[...])
    lg_r[...] = lg - jnp.where(mask_r[...], 0.0, 1e6)


def _post(a0, a1, d0, d1, x, mask, Wf, csum, W1, b1, W2, b2):
    return pl.pallas_call(
        _post_body,
        grid=(NBLK,),
        in_specs=[
            pl.BlockSpec((NB, MSGW), lambda i: (i, 0)),
            pl.BlockSpec((NB, MSGW), lambda i: (i, 0)),
            pl.BlockSpec((NB, 16), lambda i: (i, 0)),
            pl.BlockSpec((NB, 16), lambda i: (i, 0)),
            pl.BlockSpec((NB, F), lambda i: (i, 0)),
            pl.BlockSpec((NB, 1), lambda i: (i, 0)),
            pl.BlockSpec((HEADS, F, 2 * H), lambda i: (0, 0, 0)),
            pl.BlockSpec((1, 2 * H), lambda i: (0, 0)),
            pl.BlockSpec((H, 4 * H), lambda i: (0, 0)),
            pl.BlockSpec((1, H), lambda i: (0, 0)),
            pl.BlockSpec((1, H + F), lambda i: (0, 0)),
            pl.BlockSpec((1, 1), lambda i: (0, 0)),
        ],
        out_specs=pl.BlockSpec((NB, 1), lambda i: (i, 0)),
        out_shape=jax.ShapeDtypeStruct((N, 1), jnp.float32),
    )(a0, a1, d0, d1, x, mask, Wf, csum, W1, b1.reshape(1, H), W2,
      b2.reshape(1, 1))


def _sample_body(lg_r, gid_r, samp_r, logp_r):
    iota = lax.broadcasted_iota(jnp.int32, (CITIES, 1), 0)
    for g in range(G):
        l = lg_r[pl.ds(g * CITIES, CITIES), :]
        m = jnp.max(l, axis=0, keepdims=True)
        den = jnp.sum(jnp.exp(l - m), axis=0, keepdims=True)
        cand = jnp.where(l >= m, iota, N)
        samp = jnp.min(cand, axis=0, keepdims=True)
        samp_r[pl.ds(g, 1), :] = samp + gid_r[pl.ds(g, 1), :]
        logp_r[pl.ds(g, 1), :] = -jnp.log(den)


def _sample(logits, gid2d):
    return pl.pallas_call(
        _sample_body,
        out_shape=(
            jax.ShapeDtypeStruct((G, 1), jnp.int32),
            jax.ShapeDtypeStruct((G, 1), jnp.float32),
        ),
    )(logits, gid2d)


def kernel(x, edge_index, batch, center_node_index, mask, graph_id_index,
           W0, b0, initial_embed, Wg, a_src, a_dst, W1, b1, W2, b2):
    Wf, V, cc, csum = _prep_weights(W0, b0, initial_embed, Wg, a_src, a_dst)
    T, xsplit, pmax = _prep_nodes(x, V, cc)
    M = _reduce_bound(pmax.reshape(NBLK, 8))
    zden = jnp.zeros((ROWS_PER_TILE, 16), jnp.float32)
    zmsg = jnp.zeros((ROWS_PER_TILE, MSGW), jnp.float32)
    den, w4 = _sc_denominators(edge_index, T, M, zden)
    acc = _sc_messages(edge_index, xsplit.reshape(2 * N, 64),
                       w4.reshape(E * 4), zmsg)
    logits = _post(acc[0], acc[1], den[0], den[1], x, mask, Wf, csum,
                   W1, b1, W2, b2)
    samp, logp = _sample(logits, graph_id_index.reshape(G, 1))
    return samp.reshape(G), logp.reshape(G)


# speedup vs baseline: 52.7092x; 1.2015x over previous
import functools

import jax
import jax.numpy as jnp
from jax import lax
from jax.experimental import pallas as pl
from jax.experimental.pallas import tpu as pltpu
from jax.experimental.pallas import tpu_sc as plsc

N = 10000
E = 320000
F = 128
H = 128
HEADS = 3
G = 100
CITIES = N // G

MSGW = 192
CH1 = 80
EPT1 = E // 32
NCH1 = EPT1 // CH1
CH2 = 32
EPT2 = E // 16
NCH2 = EPT2 // CH2
ROWS_PER_TILE = N // 16
NB = 1000
NBLK = N // NB


def _w_body(W0_r, b0_r, ie_r, Wg_r, asrc_r, adst_r, Wf_r, V_r, cc_r, csum_r):
    csum = jnp.zeros((1, 2 * H), jnp.float32)
    vcols = []
    dcols = []
    ccs = []
    ccd = []
    for h in range(HEADS):
        Wg_top = Wg_r[h, :H, :]
        Wg_bot = Wg_r[h, H:, :]
        Wf = jnp.einsum('kf,ko->fo', W0_r[...], Wg_bot,
                        preferred_element_type=jnp.float32)
        Wf_r[h, :, :] = Wf
        const = (jnp.einsum('xk,ko->xo', ie_r[...], Wg_top,
                            preferred_element_type=jnp.float32)
                 + jnp.einsum('xk,ko->xo', b0_r[...], Wg_bot,
                              preferred_element_type=jnp.float32))
        csum = csum + const
        asr = asrc_r[h:h + 1, :]
        adr = adst_r[h:h + 1, :]
        vcols.append(jnp.einsum('fo,xo->fx', Wf, asr,
                                preferred_element_type=jnp.float32))
        dcols.append(jnp.einsum('fo,xo->fx', Wf, adr,
                                preferred_element_type=jnp.float32))
        ccs.append(jnp.einsum('xo,yo->xy', const, asr,
                              preferred_element_type=jnp.float32))
        ccd.append(jnp.einsum('xo,yo->xy', const, adr,
                              preferred_element_type=jnp.float32))
    zc = jnp.zeros((F, 2), jnp.float32)
    V_r[...] = jnp.concatenate(vcols + dcols + [zc], axis=1)
    cc_r[...] = jnp.concatenate(ccs + ccd + [jnp.zeros((1, 2), jnp.float32)],
                                axis=1)
    csum_r[...] = csum


def _prep_weights(W0, b0, initial_embed, Wg, a_src, a_dst):
    return pl.pallas_call(
        _w_body,
        out_shape=(
            jax.ShapeDtypeStruct((HEADS, F, 2 * H), jnp.float32),
            jax.ShapeDtypeStruct((F, 8), jnp.float32),
            jax.ShapeDtypeStruct((1, 8), jnp.float32),
            jax.ShapeDtypeStruct((1, 2 * H), jnp.float32),
        ),
    )(W0, b0.reshape(1, H), initial_embed.reshape(1, H), Wg, a_src, a_dst)


def _node_body(x_r, V_r, cc_r, T_r, xs_r, pmax_r):
    xv = x_r[...]
    T = jnp.dot(xv, V_r[...], preferred_element_type=jnp.float32) + cc_r[...]
    T_r[...] = T
    xs_r[0, :, :] = xv[:, :64]
    xs_r[1, :, :] = xv[:, 64:]
    pmax_r[0, :, :] = jnp.max(T, axis=0, keepdims=True)


def _prep_nodes(x, V, cc):
    return pl.pallas_call(
        _node_body,
        grid=(NBLK,),
        in_specs=[
            pl.BlockSpec((NB, F), lambda i: (i, 0)),
            pl.BlockSpec((F, 8), lambda i: (0, 0)),
            pl.BlockSpec((1, 8), lambda i: (0, 0)),
        ],
        out_specs=(
            pl.BlockSpec((NB, 8), lambda i: (i, 0)),
            pl.BlockSpec((2, NB, 64), lambda i: (0, i, 0)),
            pl.BlockSpec((1, 1, 8), lambda i: (i, 0, 0)),
        ),
        out_shape=(
            jax.ShapeDtypeStruct((N, 8), jnp.float32),
            jax.ShapeDtypeStruct((2, N, 64), jnp.float32),
            jax.ShapeDtypeStruct((NBLK, 1, 8), jnp.float32),
        ),
    )(x, V, cc)


def _bound_body(pmax_r, M_r):
    cm = jnp.max(pmax_r[...], axis=0, keepdims=True)
    r = lax.broadcasted_iota(jnp.int32, (8, 16), 0)
    c = lax.broadcasted_iota(jnp.int32, (8, 16), 1)
    sel = jnp.where((c < 3) & ((r == c) | (r == c + 3)), 1.0, 0.0)
    m = jnp.dot(cm, sel, preferred_element_type=jnp.float32)
    m = jnp.where(m > 0, m, 0.2 * m)
    M_r[...] = m


def _reduce_bound(pmax):
    return pl.pallas_call(
        _bound_body,
        out_shape=jax.ShapeDtypeStruct((1, 16), jnp.float32),
    )(pmax)


_SC_PARAMS = pltpu.CompilerParams(use_tc_tiling_on_sc=False,
                                  needs_layout_passes=False)
_LANE = lambda: lax.broadcasted_iota(jnp.int32, (16,), 0)


def _den_body(ei_hbm, T_hbm, M_hbm, z_hbm,
              out_hbm, w4_hbm,
              src_v, dst_v,
              srr0, srr1, dsc0, dsc1, dsp0, dsp1,
              tsr0, tsr1, tdr0, tdr1,
              wst0, wst1, w4st0, w4st1, M_v,
              acc_sh,
              sin, st0, st1, ss, sw):
    cid = lax.axis_index("c")
    sid = lax.axis_index("s")
    pltpu.sync_copy(M_hbm, M_v)
    pltpu.sync_copy(z_hbm,
                    acc_sh.at[pl.ds(sid * ROWS_PER_TILE, ROWS_PER_TILE)])
    zero16 = jnp.zeros((16,), jnp.float32)
    for r in range(CH1):
        wst0[r, pl.ds(0, 16)] = zero16
        wst1[r, pl.ds(0, 16)] = zero16
    plsc.subcore_barrier()
    mv = M_v[0, :]
    m0, m1, m2 = mv[0], mv[1], mv[2]

    srrs = (srr0, srr1)
    dscs = (dsc0, dsc1)
    dsps = (dsp0, dsp1)
    tsrs = (tsr0, tsr1)
    tdrs = (tdr0, tdr1)
    wsts = (wst0, wst1)
    w4sts = (w4st0, w4st1)
    sts = (st0, st1)

    def issue_in(j):
        cj = jnp.minimum(j, NCH1 - 1)
        base = (cid * 16 + sid) * EPT1 + cj * CH1
        pltpu.async_copy(ei_hbm.at[0, pl.ds(base, CH1)], src_v, sin)
        pltpu.async_copy(ei_hbm.at[1, pl.ds(base, CH1)], dst_v, sin)

    def wait_in():
        pltpu.make_async_copy(ei_hbm.at[0, pl.ds(0, CH1)], src_v, sin).wait()
        pltpu.make_async_copy(ei_hbm.at[1, pl.ds(0, CH1)], dst_v, sin).wait()

    def extract(b):
        for g in range(CH1 // 16):
            sl = pl.ds(g * 16, 16)
            srrs[b][sl] = src_v[sl]
            dscs[b][sl] = dst_v[sl]

    def issue_trows(b):
        pltpu.async_copy(T_hbm.at[srrs[b]], tsrs[b], sts[b])
        pltpu.async_copy(T_hbm.at[dscs[b]], tdrs[b], sts[b])

    def wait_trows(b):
        pltpu.make_async_copy(T_hbm.at[srrs[b]], tsrs[b], sts[b]).wait()
        pltpu.make_async_copy(T_hbm.at[dscs[b]], tdrs[b], sts[b]).wait()

    def compute(b):
        for g in range(CH1 // 16):
            grows = jnp.full((16,), g * 16, jnp.int32) + _LANE()
            c0 = jnp.full((16,), 0, jnp.int32)
            for h, m in ((0, m0), (1, m1), (2, m2)):
                es = plsc.load_gather(tsrs[b], [grows, c0 + h])
                ed = plsc.load_gather(tdrs[b], [grows, c0 + (3 + h)])
                ev = es + ed
                ev = jnp.where(ev < 0, 0.2 * ev, ev)
                w = jnp.exp(ev - jnp.full((16,), m, jnp.float32))
                plsc.store_scatter(wsts[b], [grows, c0 + h], w)
                plsc.store_scatter(w4sts[b], [grows, c0 + h], w)

    def issue_scat(j, b):
        for g in range(CH1 // 16):
            sl = pl.ds(g * 16, 16)
            dsps[b][sl] = dscs[b][sl]
        pltpu.async_copy(wsts[b], acc_sh.at[dsps[b]], ss, add=True)
        cj = jnp.minimum(j, NCH1 - 1)
        base = (cid * 16 + sid) * EPT1 + cj * CH1
        pltpu.async_copy(w4sts[b], w4_hbm.at[pl.ds(base, CH1)], sw)

    def wait_scat(b):
        pltpu.make_async_copy(wsts[b], acc_sh.at[dsps[b]], ss).wait()
        pltpu.make_async_copy(w4sts[b], w4_hbm.at[pl.ds(0, CH1)], sw).wait()

    def body(j, b, do_wait_scat):
        nb = 1 - b
        wait_in()
        extract(nb)
        issue_in(j + 2)
        issue_trows(nb)
        wait_trows(b)
        if do_wait_scat:
            wait_scat(b)
        compute(b)
        issue_scat(j, b)

    issue_in(0)
    wait_in()
    extract(0)
    issue_trows(0)
    issue_in(1)

    body(0, 0, False)
    body(1, 1, False)
    body(2, 0, True)

    def two(j):
        body(j, 1, True)
        body(j + 1, 0, True)

    pl.loop(3, NCH1, step=2)(two)

    wait_scat((NCH1 - 1) % 2)
    wait_scat(NCH1 % 2)
    wait_trows(NCH1 % 2)
    wait_in()

    plsc.subcore_barrier()
    pltpu.sync_copy(
        acc_sh.at[pl.ds(sid * ROWS_PER_TILE, ROWS_PER_TILE)],
        out_hbm.at[cid, pl.ds(sid * ROWS_PER_TILE, ROWS_PER_TILE)])


def _sc_denominators(edge_index, T, M, zrows16):
    mesh = plsc.VectorSubcoreMesh(core_axis_name="c", subcore_axis_name="s")
    f = pl.kernel(
        _den_body,
        out_type=(jax.ShapeDtypeStruct((2, N, 16), jnp.float32),
                  jax.ShapeDtypeStruct((E, 4), jnp.float32)),
        mesh=mesh,
        compiler_params=_SC_PARAMS,
        scratch_types=[
            pltpu.VMEM((CH1,), jnp.int32),
            pltpu.VMEM((CH1,), jnp.int32),
            pltpu.VMEM((CH1,), jnp.int32),
            pltpu.VMEM((CH1,), jnp.int32),
            pltpu.VMEM((CH1,), jnp.int32),
            pltpu.VMEM((CH1,), jnp.int32),
            pltpu.VMEM((CH1,), jnp.int32),
            pltpu.VMEM((CH1,), jnp.int32),
            pltpu.VMEM((CH1, 8), jnp.float32),
            pltpu.VMEM((CH1, 8), jnp.float32),
            pltpu.VMEM((CH1, 8), jnp.float32),
            pltpu.VMEM((CH1, 8), jnp.float32),
            pltpu.VMEM((CH1, 16), jnp.float32),
            pltpu.VMEM((CH1, 16), jnp.float32),
            pltpu.VMEM((CH1, 4), jnp.float32),
            pltpu.VMEM((CH1, 4), jnp.float32),
            pltpu.VMEM((1, 16), jnp.float32),
            pltpu.VMEM_SHARED((N, 16), jnp.float32),
            pltpu.SemaphoreType.DMA,
            pltpu.SemaphoreType.DMA,
            pltpu.SemaphoreType.DMA,
            pltpu.SemaphoreType.DMA,
            pltpu.SemaphoreType.DMA,
        ],
    )
    return f(edge_index, T, M, zrows16)


def _msg_body(ei_hbm, xf_hbm, w4_hbm, z_hbm,
              out_hbm,
              src_v, dst_v, w4_v,
              idx0, idx1, dsc0, dsc1, dsp0, dsp1, w4c0, w4c1,
              rows0, rows1, stag_v,
              acc_sh,
              sin, sr0, sr1, ss):
    cid = lax.axis_index("c")
    sid = lax.axis_index("s")
    pltpu.sync_copy(z_hbm,
                    acc_sh.at[pl.ds(sid * ROWS_PER_TILE, ROWS_PER_TILE)])
    plsc.subcore_barrier()
    coff = cid * N

    idxs = (idx0, idx1)
    dscs = (dsc0, dsc1)
    dsps = (dsp0, dsp1)
    w4cs = (w4c0, w4c1)
    rowss = (rows0, rows1)
    srs = (sr0, sr1)

    def issue_in(j):
        cj = jnp.minimum(j, NCH2 - 1)
        base = sid * EPT2 + cj * CH2
        pltpu.async_copy(ei_hbm.at[0, pl.ds(base, CH2)], src_v, sin)
        pltpu.async_copy(ei_hbm.at[1, pl.ds(base, CH2)], dst_v, sin)
        pltpu.async_copy(w4_hbm.at[pl.ds(base * 4, CH2 * 4)], w4_v, sin)

    def wait_in():
        pltpu.make_async_copy(ei_hbm.at[0, pl.ds(0, CH2)], src_v, sin).wait()
        pltpu.make_async_copy(ei_hbm.at[1, pl.ds(0, CH2)], dst_v, sin).wait()
        pltpu.make_async_copy(w4_hbm.at[pl.ds(0, CH2 * 4)], w4_v, sin).wait()

    def extract(b):
        for g in range(CH2 // 16):
            sl = pl.ds(g * 16, 16)
            idxs[b][sl] = src_v[sl] + jnp.full((16,), coff, jnp.int32)
            dscs[b][sl] = dst_v[sl]
        for g in range(CH2 * 4 // 16):
            sl = pl.ds(g * 16, 16)
            w4cs[b][sl] = w4_v[sl]

    def issue_rows(b):
        pltpu.async_copy(xf_hbm.at[idxs[b]], rowss[b], srs[b])

    def wait_rows(b):
        pltpu.make_async_copy(xf_hbm.at[idxs[b]], rowss[b], srs[b]).wait()

    def issue_scat(b):
        for g in range(CH2 // 16):
            sl = pl.ds(g * 16, 16)
            dsps[b][sl] = dscs[b][sl]
        pltpu.async_copy(stag_v, acc_sh.at[dsps[b]], ss, add=True)

    def wait_scat(b):
        pltpu.make_async_copy(stag_v, acc_sh.at[dsps[b]], ss).wait()

    def compute(b):
        w4c = w4cs[b]
        rows = rowss[b]
        for g in range(CH2 // 16):
            r4 = (jnp.full((16,), g * 16, jnp.int32) + _LANE()) * 4
            wv0 = plsc.load_gather(w4c, [r4])
            wv1 = plsc.load_gather(w4c, [r4 + 1])
            wv2 = plsc.load_gather(w4c, [r4 + 2])
            for i in range(16):
                e = g * 16 + i
                wb0 = jnp.full((16,), wv0[i], jnp.float32)
                wb1 = jnp.full((16,), wv1[i], jnp.float32)
                wb2 = jnp.full((16,), wv2[i], jnp.float32)
                for cb in range(4):
                    xvv = rows[e, pl.ds(cb * 16, 16)]
                    stag_v[e, pl.ds(cb * 16, 16)] = xvv * wb0
                    stag_v[e, pl.ds(64 + cb * 16, 16)] = xvv * wb1
                    stag_v[e, pl.ds(128 + cb * 16, 16)] = xvv * wb2

    def body(j, b, first):
        nb = 1 - b
        wait_in()
        extract(nb)
        issue_in(j + 2)
        issue_rows(nb)
        wait_rows(b)
        if first:
            @pl.when(j > 0)
            def _():
                wait_scat(nb)
        else:
            wait_scat(nb)
        compute(b)
        issue_scat(b)

    issue_in(0)
    wait_in()
    extract(0)
    issue_rows(0)
    issue_in(1)

    def two(j):
        body(j, 0, True)
        body(j + 1, 1, False)

    pl.loop(0, NCH2 - 1, step=2)(two)
    body(NCH2 - 1, (NCH2 - 1) % 2, False)

    wait_scat((NCH2 - 1) % 2)
    wait_rows(NCH2 % 2)
    wait_in()

    plsc.subcore_barrier()
    pltpu.sync_copy(
        acc_sh.at[pl.ds(sid * ROWS_PER_TILE, ROWS_PER_TILE)],
        out_hbm.at[cid, pl.ds(sid * ROWS_PER_TILE, ROWS_PER_TILE)])


def _sc_messages(edge_index, xflat, w4flat, zrows):
    mesh = plsc.VectorSubcoreMesh(core_axis_name="c", subcore_axis_name="s")
    f = pl.kernel(
        _msg_body,
        out_type=jax.ShapeDtypeStruct((2, N, MSGW), jnp.float32),
        mesh=mesh,
        compiler_params=_SC_PARAMS,
        scratch_types=[
            pltpu.VMEM((CH2,), jnp.int32),
            pltpu.VMEM((CH2,), jnp.int32),
            pltpu.VMEM((CH2 * 4,), jnp.float32),
            pltpu.VMEM((CH2,), jnp.int32),
            pltpu.VMEM((CH2,), jnp.int32),
            pltpu.VMEM((CH2,), jnp.int32),
            pltpu.VMEM((CH2,), jnp.int32),
            pltpu.VMEM((CH2,), jnp.int32),
            pltpu.VMEM((CH2,), jnp.int32),
            pltpu.VMEM((CH2 * 4,), jnp.float32),
            pltpu.VMEM((CH2 * 4,), jnp.float32),
            pltpu.VMEM((CH2, 64), jnp.float32),
            pltpu.VMEM((CH2, 64), jnp.float32),
            pltpu.VMEM((CH2, MSGW), jnp.float32),
            pltpu.VMEM_SHARED((N, MSGW), jnp.float32),
            pltpu.SemaphoreType.DMA,
            pltpu.SemaphoreType.DMA,
            pltpu.SemaphoreType.DMA,
            pltpu.SemaphoreType.DMA,
        ],
    )
    return f(edge_index, xflat, w4flat, zrows)


def _post_body(a0_r, a1_r, d0_r, d1_r, x_r, mask_r, Wf_r, csum_r, W1_r, b1_r,
               W2_r, b2_r, lg_r):
    den3 = d0_r[...] + d1_r[...]
    has = den3[:, 0:1] > 0
    osum = jnp.where(has, csum_r[...], 0.0)
    for h in range(HEADS):
        den = den3[:, h:h + 1]
        agg = jnp.concatenate(
            [a0_r[:, h * 64:(h + 1) * 64], a1_r[:, h * 64:(h + 1) * 64]],
            axis=1)
        agg = jnp.where(den > 0, agg / jnp.where(den > 0, den, 1.0), 0.0)
        osum = osum + jnp.dot(agg, Wf_r[h, :, :],
                              preferred_element_type=jnp.float32)
    enc = osum * (1.0 / 3.0)
    enc = jnp.where(enc > 0, enc, jnp.exp(jnp.minimum(enc, 0.0)) - 1.0)
    encg = enc.reshape(NB // CITIES, CITIES, 2 * H)
    center = jnp.broadcast_to(encg[:, 0:1, :], encg.shape)
    crep = center.reshape(NB, 2 * H)
    h1 = (jnp.einsum('nc,kc->nk', crep, W1_r[:, :2 * H],
                     preferred_element_type=jnp.float32)
          + jnp.einsum('nc,kc->nk', enc, W1_r[:, 2 * H:],
                       preferred_element_type=jnp.float32)
          + b1_r[...])
    h1 = jnp.maximum(h1, 0.0)
    lg = (jnp.einsum('nf,of->no', x_r[...], W2_r[:, :F],
                     preferred_element_type=jnp.float32)
          + jnp.einsum('nh,oh->no', h1, W2_r[:, F:],
                       preferred_element_type=jnp.float32)
          + b2_r[...])
    lg_r[...] = lg - jnp.where(mask_r[...], 0.0, 1e6)


def _post(a0, a1, d0, d1, x, mask, Wf, csum, W1, b1, W2, b2):
    return pl.pallas_call(
        _post_body,
        grid=(NBLK,),
        in_specs=[
            pl.BlockSpec((NB, MSGW), lambda i: (i, 0)),
            pl.BlockSpec((NB, MSGW), lambda i: (i, 0)),
            pl.BlockSpec((NB, 16), lambda i: (i, 0)),
            pl.BlockSpec((NB, 16), lambda i: (i, 0)),
            pl.BlockSpec((NB, F), lambda i: (i, 0)),
            pl.BlockSpec((NB, 1), lambda i: (i, 0)),
            pl.BlockSpec((HEADS, F, 2 * H), lambda i: (0, 0, 0)),
            pl.BlockSpec((1, 2 * H), lambda i: (0, 0)),
            pl.BlockSpec((H, 4 * H), lambda i: (0, 0)),
            pl.BlockSpec((1, H), lambda i: (0, 0)),
            pl.BlockSpec((1, H + F), lambda i: (0, 0)),
            pl.BlockSpec((1, 1), lambda i: (0, 0)),
        ],
        out_specs=pl.BlockSpec((NB, 1), lambda i: (i, 0)),
        out_shape=jax.ShapeDtypeStruct((N, 1), jnp.float32),
    )(a0, a1, d0, d1, x, mask, Wf, csum, W1, b1.reshape(1, H), W2,
      b2.reshape(1, 1))


def _sample_body(lg_r, gid_r, samp_r, logp_r):
    iota = lax.broadcasted_iota(jnp.int32, (CITIES, 1), 0)
    for g in range(G):
        l = lg_r[pl.ds(g * CITIES, CITIES), :]
        m = jnp.max(l, axis=0, keepdims=True)
        den = jnp.sum(jnp.exp(l - m), axis=0, keepdims=True)
        cand = jnp.where(l >= m, iota, N)
        samp = jnp.min(cand, axis=0, keepdims=True)
        samp_r[pl.ds(g, 1), :] = samp + gid_r[pl.ds(g, 1), :]
        logp_r[pl.ds(g, 1), :] = -jnp.log(den)


def _sample(logits, gid2d):
    return pl.pallas_call(
        _sample_body,
        out_shape=(
            jax.ShapeDtypeStruct((G, 1), jnp.int32),
            jax.ShapeDtypeStruct((G, 1), jnp.float32),
        ),
    )(logits, gid2d)


def kernel(x, edge_index, batch, center_node_index, mask, graph_id_index,
           W0, b0, initial_embed, Wg, a_src, a_dst, W1, b1, W2, b2):
    Wf, V, cc, csum = _prep_weights(W0, b0, initial_embed, Wg, a_src, a_dst)
    T, xsplit, pmax = _prep_nodes(x, V, cc)
    M = _reduce_bound(pmax.reshape(NBLK, 8))
    zden = jnp.zeros((ROWS_PER_TILE, 16), jnp.float32)
    zmsg = jnp.zeros((ROWS_PER_TILE, MSGW), jnp.float32)
    den, w4 = _sc_denominators(edge_index, T, M, zden)
    acc = _sc_messages(edge_index, xsplit.reshape(2 * N, 64),
                       w4.reshape(E * 4), zmsg)
    logits = _post(acc[0], acc[1], den[0], den[1], x, mask, Wf, csum,
                   W1, b1, W2, b2)
    samp, logp = _sample(logits, graph_id_index.reshape(G, 1))
    return samp.reshape(G), logp.reshape(G)


# speedup vs baseline: 53.1947x; 1.0092x over previous
import functools

import jax
import jax.numpy as jnp
from jax import lax
from jax.experimental import pallas as pl
from jax.experimental.pallas import tpu as pltpu
from jax.experimental.pallas import tpu_sc as plsc

N = 10000
E = 320000
F = 128
H = 128
HEADS = 3
G = 100
CITIES = N // G

MSGW = 192
CH1 = 80
EPT1 = E // 32
NCH1 = EPT1 // CH1
CH2 = 32
EPT2 = E // 16
NCH2 = EPT2 // CH2
ROWS_PER_TILE = N // 16
NB = 1000
NBLK = N // NB


def _a_body(W0_r, b0_r, ie_r, Wg_r, asrc_r, adst_r, x_r,
            Wf_r, csum_r, T_r, xs_r, M_r, V_s, cc_s):
    i = pl.program_id(0)

    @pl.when(i == 0)
    def _():
        csum = jnp.zeros((1, 2 * H), jnp.float32)
        vcols = []
        dcols = []
        ccs = []
        ccd = []
        for h in range(HEADS):
            Wg_top = Wg_r[h, :H, :]
            Wg_bot = Wg_r[h, H:, :]
            Wf = jnp.einsum('kf,ko->fo', W0_r[...], Wg_bot,
                            preferred_element_type=jnp.float32)
            Wf_r[h, :, :] = Wf
            const = (jnp.einsum('xk,ko->xo', ie_r[...], Wg_top,
                                preferred_element_type=jnp.float32)
                     + jnp.einsum('xk,ko->xo', b0_r[...], Wg_bot,
                                  preferred_element_type=jnp.float32))
            csum = csum + const
            asr = asrc_r[h:h + 1, :]
            adr = adst_r[h:h + 1, :]
            vcols.append(jnp.einsum('fo,xo->fx', Wf, asr,
                                    preferred_element_type=jnp.float32))
            dcols.append(jnp.einsum('fo,xo->fx', Wf, adr,
                                    preferred_element_type=jnp.float32))
            ccs.append(jnp.einsum('xo,yo->xy', const, asr,
                                  preferred_element_type=jnp.float32))
            ccd.append(jnp.einsum('xo,yo->xy', const, adr,
                                  preferred_element_type=jnp.float32))
        zc = jnp.zeros((F, 2), jnp.float32)
        V_s[...] = jnp.concatenate(vcols + dcols + [zc], axis=1)
        cc_s[...] = jnp.concatenate(ccs + ccd
                                    + [jnp.zeros((1, 2), jnp.float32)], axis=1)
        csum_r[...] = csum

    xv = x_r[...]
    T = jnp.dot(xv, V_s[...], preferred_element_type=jnp.float32) + cc_s[...]
    T_r[...] = T
    xs_r[0, :, :] = xv[:, :64]
    xs_r[1, :, :] = xv[:, 64:]
    bm = jnp.max(T, axis=0, keepdims=True)

    @pl.when(i == 0)
    def _():
        M_r[:, :8] = bm
        M_r[:, 8:] = bm

    @pl.when(i > 0)
    def _():
        M_r[:, :8] = jnp.maximum(M_r[:, :8], bm)

    @pl.when(i == NBLK - 1)
    def _():
        cm = M_r[:, :8]
        r = lax.broadcasted_iota(jnp.int32, (8, 16), 0)
        c = lax.broadcasted_iota(jnp.int32, (8, 16), 1)
        sel = jnp.where((c < 3) & ((r == c) | (r == c + 3)), 1.0, 0.0)
        m = jnp.dot(cm, sel, preferred_element_type=jnp.float32)
        M_r[...] = jnp.where(m > 0, m, 0.2 * m)


def _prep(x, W0, b0, initial_embed, Wg, a_src, a_dst):
    return pl.pallas_call(
        _a_body,
        grid=(NBLK,),
        in_specs=[
            pl.BlockSpec((F, F), lambda i: (0, 0)),
            pl.BlockSpec((1, H), lambda i: (0, 0)),
            pl.BlockSpec((1, H), lambda i: (0, 0)),
            pl.BlockSpec((HEADS, 2 * H, 2 * H), lambda i: (0, 0, 0)),
            pl.BlockSpec((HEADS, 2 * H), lambda i: (0, 0)),
            pl.BlockSpec((HEADS, 2 * H), lambda i: (0, 0)),
            pl.BlockSpec((NB, F), lambda i: (i, 0)),
        ],
        out_specs=(
            pl.BlockSpec((HEADS, F, 2 * H), lambda i: (0, 0, 0)),
            pl.BlockSpec((1, 2 * H), lambda i: (0, 0)),
            pl.BlockSpec((NB, 8), lambda i: (i, 0)),
            pl.BlockSpec((2, NB, 64), lambda i: (0, i, 0)),
            pl.BlockSpec((1, 16), lambda i: (0, 0)),
        ),
        out_shape=(
            jax.ShapeDtypeStruct((HEADS, F, 2 * H), jnp.float32),
            jax.ShapeDtypeStruct((1, 2 * H), jnp.float32),
            jax.ShapeDtypeStruct((N, 8), jnp.float32),
            jax.ShapeDtypeStruct((2, N, 64), jnp.float32),
            jax.ShapeDtypeStruct((1, 16), jnp.float32),
        ),
        scratch_shapes=[
            pltpu.VMEM((F, 8), jnp.float32),
            pltpu.VMEM((1, 8), jnp.float32),
        ],
    )(W0, b0.reshape(1, H), initial_embed.reshape(1, H), Wg, a_src, a_dst, x)


_SC_PARAMS = pltpu.CompilerParams(use_tc_tiling_on_sc=False,
                                  needs_layout_passes=False)
_LANE = lambda: lax.broadcasted_iota(jnp.int32, (16,), 0)


def _den_body(ei_hbm, T_hbm, M_hbm, z_hbm,
              out_hbm, w4_hbm,
              src_v, dst_v,
              srr0, srr1, dsc0, dsc1, dsp0, dsp1,
              tsr0, tsr1, tdr0, tdr1,
              wst0, wst1, w4st0, w4st1, M_v,
              acc_sh,
              sin, st0, st1, ss, sw):
    cid = lax.axis_index("c")
    sid = lax.axis_index("s")
    pltpu.sync_copy(M_hbm, M_v)
    pltpu.sync_copy(z_hbm,
                    acc_sh.at[pl.ds(sid * ROWS_PER_TILE, ROWS_PER_TILE)])
    zero16 = jnp.zeros((16,), jnp.float32)
    for r in range(CH1):
        wst0[r, pl.ds(0, 16)] = zero16
        wst1[r, pl.ds(0, 16)] = zero16
    plsc.subcore_barrier()
    mv = M_v[0, :]
    m0, m1, m2 = mv[0], mv[1], mv[2]

    srrs = (srr0, srr1)
    dscs = (dsc0, dsc1)
    dsps = (dsp0, dsp1)
    tsrs = (tsr0, tsr1)
    tdrs = (tdr0, tdr1)
    wsts = (wst0, wst1)
    w4sts = (w4st0, w4st1)
    sts = (st0, st1)

    def issue_in(j):
        cj = jnp.minimum(j, NCH1 - 1)
        base = (cid * 16 + sid) * EPT1 + cj * CH1
        pltpu.async_copy(ei_hbm.at[0, pl.ds(base, CH1)], src_v, sin)
        pltpu.async_copy(ei_hbm.at[1, pl.ds(base, CH1)], dst_v, sin)

    def wait_in():
        pltpu.make_async_copy(ei_hbm.at[0, pl.ds(0, CH1)], src_v, sin).wait()
        pltpu.make_async_copy(ei_hbm.at[1, pl.ds(0, CH1)], dst_v, sin).wait()

    def extract(b):
        for g in range(CH1 // 16):
            sl = pl.ds(g * 16, 16)
            srrs[b][sl] = src_v[sl]
            dscs[b][sl] = dst_v[sl]

    def issue_trows(b):
        pltpu.async_copy(T_hbm.at[srrs[b]], tsrs[b], sts[b])
        pltpu.async_copy(T_hbm.at[dscs[b]], tdrs[b], sts[b])

    def wait_trows(b):
        pltpu.make_async_copy(T_hbm.at[srrs[b]], tsrs[b], sts[b]).wait()
        pltpu.make_async_copy(T_hbm.at[dscs[b]], tdrs[b], sts[b]).wait()

    def compute(b):
        for g in range(CH1 // 16):
            grows = jnp.full((16,), g * 16, jnp.int32) + _LANE()
            c0 = jnp.full((16,), 0, jnp.int32)
            for h, m in ((0, m0), (1, m1), (2, m2)):
                es = plsc.load_gather(tsrs[b], [grows, c0 + h])
                ed = plsc.load_gather(tdrs[b], [grows, c0 + (3 + h)])
                ev = es + ed
                ev = jnp.where(ev < 0, 0.2 * ev, ev)
                w = jnp.exp(ev - jnp.full((16,), m, jnp.float32))
                plsc.store_scatter(wsts[b], [grows, c0 + h], w)
                plsc.store_scatter(w4sts[b], [grows, c0 + h], w)

    def issue_scat(j, b):
        for g in range(CH1 // 16):
            sl = pl.ds(g * 16, 16)
            dsps[b][sl] = dscs[b][sl]
        pltpu.async_copy(wsts[b], acc_sh.at[dsps[b]], ss, add=True)
        cj = jnp.minimum(j, NCH1 - 1)
        base = (cid * 16 + sid) * EPT1 + cj * CH1
        pltpu.async_copy(w4sts[b], w4_hbm.at[pl.ds(base, CH1)], sw)

    def wait_scat(b):
        pltpu.make_async_copy(wsts[b], acc_sh.at[dsps[b]], ss).wait()
        pltpu.make_async_copy(w4sts[b], w4_hbm.at[pl.ds(0, CH1)], sw).wait()

    def body(j, b, do_wait_scat):
        nb = 1 - b
        wait_in()
        extract(nb)
        issue_in(j + 2)
        issue_trows(nb)
        wait_trows(b)
        if do_wait_scat:
            wait_scat(b)
        compute(b)
        issue_scat(j, b)

    issue_in(0)
    wait_in()
    extract(0)
    issue_trows(0)
    issue_in(1)

    body(0, 0, False)
    body(1, 1, False)
    body(2, 0, True)

    def two(j):
        body(j, 1, True)
        body(j + 1, 0, True)

    pl.loop(3, NCH1, step=2)(two)

    wait_scat((NCH1 - 1) % 2)
    wait_scat(NCH1 % 2)
    wait_trows(NCH1 % 2)
    wait_in()

    plsc.subcore_barrier()
    pltpu.sync_copy(
        acc_sh.at[pl.ds(sid * ROWS_PER_TILE, ROWS_PER_TILE)],
        out_hbm.at[cid, pl.ds(sid * ROWS_PER_TILE, ROWS_PER_TILE)])


def _sc_denominators(edge_index, T, M, zrows16):
    mesh = plsc.VectorSubcoreMesh(core_axis_name="c", subcore_axis_name="s")
    f = pl.kernel(
        _den_body,
        out_type=(jax.ShapeDtypeStruct((2, N, 16), jnp.float32),
                  jax.ShapeDtypeStruct((E, 4), jnp.float32)),
        mesh=mesh,
        compiler_params=_SC_PARAMS,
        scratch_types=[
            pltpu.VMEM((CH1,), jnp.int32),
            pltpu.VMEM((CH1,), jnp.int32),
            pltpu.VMEM((CH1,), jnp.int32),
            pltpu.VMEM((CH1,), jnp.int32),
            pltpu.VMEM((CH1,), jnp.int32),
            pltpu.VMEM((CH1,), jnp.int32),
            pltpu.VMEM((CH1,), jnp.int32),
            pltpu.VMEM((CH1,), jnp.int32),
            pltpu.VMEM((CH1, 8), jnp.float32),
            pltpu.VMEM((CH1, 8), jnp.float32),
            pltpu.VMEM((CH1, 8), jnp.float32),
            pltpu.VMEM((CH1, 8), jnp.float32),
            pltpu.VMEM((CH1, 16), jnp.float32),
            pltpu.VMEM((CH1, 16), jnp.float32),
            pltpu.VMEM((CH1, 4), jnp.float32),
            pltpu.VMEM((CH1, 4), jnp.float32),
            pltpu.VMEM((1, 16), jnp.float32),
            pltpu.VMEM_SHARED((N, 16), jnp.float32),
            pltpu.SemaphoreType.DMA,
            pltpu.SemaphoreType.DMA,
            pltpu.SemaphoreType.DMA,
            pltpu.SemaphoreType.DMA,
            pltpu.SemaphoreType.DMA,
        ],
    )
    return f(edge_index, T, M, zrows16)


def _msg_body(ei_hbm, xf_hbm, w4_hbm, z_hbm,
              out_hbm,
              src_v, dst_v, w4_v,
              idx0, idx1, dsc0, dsc1, dsp0, dsp1, w4c0, w4c1,
              rows0, rows1, stag_v,
              acc_sh,
              sin, sr0, sr1, ss):
    cid = lax.axis_index("c")
    sid = lax.axis_index("s")
    pltpu.sync_copy(z_hbm,
                    acc_sh.at[pl.ds(sid * ROWS_PER_TILE, ROWS_PER_TILE)])
    plsc.subcore_barrier()
    coff = cid * N

    idxs = (idx0, idx1)
    dscs = (dsc0, dsc1)
    dsps = (dsp0, dsp1)
    w4cs = (w4c0, w4c1)
    rowss = (rows0, rows1)
    srs = (sr0, sr1)

    def issue_in(j):
        cj = jnp.minimum(j, NCH2 - 1)
        base = sid * EPT2 + cj * CH2
        pltpu.async_copy(ei_hbm.at[0, pl.ds(base, CH2)], src_v, sin)
        pltpu.async_copy(ei_hbm.at[1, pl.ds(base, CH2)], dst_v, sin)
        pltpu.async_copy(w4_hbm.at[pl.ds(base * 4, CH2 * 4)], w4_v, sin)

    def wait_in():
        pltpu.make_async_copy(ei_hbm.at[0, pl.ds(0, CH2)], src_v, sin).wait()
        pltpu.make_async_copy(ei_hbm.at[1, pl.ds(0, CH2)], dst_v, sin).wait()
        pltpu.make_async_copy(w4_hbm.at[pl.ds(0, CH2 * 4)], w4_v, sin).wait()

    def extract(b):
        for g in range(CH2 // 16):
            sl = pl.ds(g * 16, 16)
            idxs[b][sl] = src_v[sl] + jnp.full((16,), coff, jnp.int32)
            dscs[b][sl] = dst_v[sl]
        for g in range(CH2 * 4 // 16):
            sl = pl.ds(g * 16, 16)
            w4cs[b][sl] = w4_v[sl]

    def issue_rows(b):
        pltpu.async_copy(xf_hbm.at[idxs[b]], rowss[b], srs[b])

    def wait_rows(b):
        pltpu.make_async_copy(xf_hbm.at[idxs[b]], rowss[b], srs[b]).wait()

    def issue_scat(b):
        for g in range(CH2 // 16):
            sl = pl.ds(g * 16, 16)
            dsps[b][sl] = dscs[b][sl]
        pltpu.async_copy(stag_v, acc_sh.at[dsps[b]], ss, add=True)

    def wait_scat(b):
        pltpu.make_async_copy(stag_v, acc_sh.at[dsps[b]], ss).wait()

    def compute(b):
        w4c = w4cs[b]
        rows = rowss[b]
        for g in range(CH2 // 16):
            r4 = (jnp.full((16,), g * 16, jnp.int32) + _LANE()) * 4
            wv0 = plsc.load_gather(w4c, [r4])
            wv1 = plsc.load_gather(w4c, [r4 + 1])
            wv2 = plsc.load_gather(w4c, [r4 + 2])
            for i in range(16):
                e = g * 16 + i
                wb0 = jnp.full((16,), wv0[i], jnp.float32)
                wb1 = jnp.full((16,), wv1[i], jnp.float32)
                wb2 = jnp.full((16,), wv2[i], jnp.float32)
                for cb in range(4):
                    xvv = rows[e, pl.ds(cb * 16, 16)]
                    stag_v[e, pl.ds(cb * 16, 16)] = xvv * wb0
                    stag_v[e, pl.ds(64 + cb * 16, 16)] = xvv * wb1
                    stag_v[e, pl.ds(128 + cb * 16, 16)] = xvv * wb2

    def body(j, b, first):
        nb = 1 - b
        wait_in()
        extract(nb)
        issue_in(j + 2)
        issue_rows(nb)
        wait_rows(b)
        if first:
            @pl.when(j > 0)
            def _():
                wait_scat(nb)
        else:
            wait_scat(nb)
        compute(b)
        issue_scat(b)

    issue_in(0)
    wait_in()
    extract(0)
    issue_rows(0)
    issue_in(1)

    def two(j):
        body(j, 0, True)
        body(j + 1, 1, False)

    pl.loop(0, NCH2 - 1, step=2)(two)
    body(NCH2 - 1, (NCH2 - 1) % 2, False)

    wait_scat((NCH2 - 1) % 2)
    wait_rows(NCH2 % 2)
    wait_in()

    plsc.subcore_barrier()
    pltpu.sync_copy(
        acc_sh.at[pl.ds(sid * ROWS_PER_TILE, ROWS_PER_TILE)],
        out_hbm.at[cid, pl.ds(sid * ROWS_PER_TILE, ROWS_PER_TILE)])


def _sc_messages(edge_index, xflat, w4flat, zrows):
    mesh = plsc.VectorSubcoreMesh(core_axis_name="c", subcore_axis_name="s")
    f = pl.kernel(
        _msg_body,
        out_type=jax.ShapeDtypeStruct((2, N, MSGW), jnp.float32),
        mesh=mesh,
        compiler_params=_SC_PARAMS,
        scratch_types=[
            pltpu.VMEM((CH2,), jnp.int32),
            pltpu.VMEM((CH2,), jnp.int32),
            pltpu.VMEM((CH2 * 4,), jnp.float32),
            pltpu.VMEM((CH2,), jnp.int32),
            pltpu.VMEM((CH2,), jnp.int32),
            pltpu.VMEM((CH2,), jnp.int32),
            pltpu.VMEM((CH2,), jnp.int32),
            pltpu.VMEM((CH2,), jnp.int32),
            pltpu.VMEM((CH2,), jnp.int32),
            pltpu.VMEM((CH2 * 4,), jnp.float32),
            pltpu.VMEM((CH2 * 4,), jnp.float32),
            pltpu.VMEM((CH2, 64), jnp.float32),
            pltpu.VMEM((CH2, 64), jnp.float32),
            pltpu.VMEM((CH2, MSGW), jnp.float32),
            pltpu.VMEM_SHARED((N, MSGW), jnp.float32),
            pltpu.SemaphoreType.DMA,
            pltpu.SemaphoreType.DMA,
            pltpu.SemaphoreType.DMA,
            pltpu.SemaphoreType.DMA,
        ],
    )
    return f(edge_index, xflat, w4flat, zrows)


def _post_body(a0_r, a1_r, d0_r, d1_r, x_r, mask_r, Wf_r, csum_r, W1_r, b1_r,
               W2_r, b2_r, gid_r, samp_r, logp_r):
    den3 = d0_r[...] + d1_r[...]
    has = den3[:, 0:1] > 0
    osum = jnp.where(has, csum_r[...], 0.0)
    for h in range(HEADS):
        den = den3[:, h:h + 1]
        agg = jnp.concatenate(
            [a0_r[:, h * 64:(h + 1) * 64], a1_r[:, h * 64:(h + 1) * 64]],
            axis=1)
        agg = jnp.where(den > 0, agg / jnp.where(den > 0, den, 1.0), 0.0)
        osum = osum + jnp.dot(agg, Wf_r[h, :, :],
                              preferred_element_type=jnp.float32)
    enc = osum * (1.0 / 3.0)
    enc = jnp.where(enc > 0, enc, jnp.exp(jnp.minimum(enc, 0.0)) - 1.0)
    encg = enc.reshape(NB // CITIES, CITIES, 2 * H)
    center = jnp.broadcast_to(encg[:, 0:1, :], encg.shape)
    crep = center.reshape(NB, 2 * H)
    h1 = (jnp.einsum('nc,kc->nk', crep, W1_r[:, :2 * H],
                     preferred_element_type=jnp.float32)
          + jnp.einsum('nc,kc->nk', enc, W1_r[:, 2 * H:],
                       preferred_element_type=jnp.float32)
          + b1_r[...])
    h1 = jnp.maximum(h1, 0.0)
    lg = (jnp.einsum('nf,of->no', x_r[...], W2_r[:, :F],
                     preferred_element_type=jnp.float32)
          + jnp.einsum('nh,oh->no', h1, W2_r[:, F:],
                       preferred_element_type=jnp.float32)
          + b2_r[...])
    lg = lg - jnp.where(mask_r[...], 0.0, 1e6)
    iota = lax.broadcasted_iota(jnp.int32, (CITIES, 1), 0)
    for g in range(NB // CITIES):
        l = lg[g * CITIES:(g + 1) * CITIES, :]
        m = jnp.max(l, axis=0, keepdims=True)
        den = jnp.sum(jnp.exp(l - m), axis=0, keepdims=True)
        cand = jnp.where(l >= m, iota, N)
        samp = jnp.min(cand, axis=0, keepdims=True)
        samp_r[0, pl.ds(g, 1), :] = samp + gid_r[0, pl.ds(g, 1), :]
        logp_r[0, pl.ds(g, 1), :] = -jnp.log(den)


def _post(a0, a1, d0, d1, x, mask, Wf, csum, W1, b1, W2, b2, gid2d):
    return pl.pallas_call(
        _post_body,
        grid=(NBLK,),
        in_specs=[
            pl.BlockSpec((NB, MSGW), lambda i: (i, 0)),
            pl.BlockSpec((NB, MSGW), lambda i: (i, 0)),
            pl.BlockSpec((NB, 16), lambda i: (i, 0)),
            pl.BlockSpec((NB, 16), lambda i: (i, 0)),
            pl.BlockSpec((NB, F), lambda i: (i, 0)),
            pl.BlockSpec((NB, 1), lambda i: (i, 0)),
            pl.BlockSpec((HEADS, F, 2 * H), lambda i: (0, 0, 0)),
            pl.BlockSpec((1, 2 * H), lambda i: (0, 0)),
            pl.BlockSpec((H, 4 * H), lambda i: (0, 0)),
            pl.BlockSpec((1, H), lambda i: (0, 0)),
            pl.BlockSpec((1, H + F), lambda i: (0, 0)),
            pl.BlockSpec((1, 1), lambda i: (0, 0)),
            pl.BlockSpec((1, G // NBLK, 1), lambda i: (i, 0, 0)),
        ],
        out_specs=(
            pl.BlockSpec((1, G // NBLK, 1), lambda i: (i, 0, 0)),
            pl.BlockSpec((1, G // NBLK, 1), lambda i: (i, 0, 0)),
        ),
        out_shape=(
            jax.ShapeDtypeStruct((NBLK, G // NBLK, 1), jnp.int32),
            jax.ShapeDtypeStruct((NBLK, G // NBLK, 1), jnp.float32),
        ),
    )(a0, a1, d0, d1, x, mask, Wf, csum, W1, b1.reshape(1, H), W2,
      b2.reshape(1, 1), gid2d)


def kernel(x, edge_index, batch, center_node_index, mask, graph_id_index,
           W0, b0, initial_embed, Wg, a_src, a_dst, W1, b1, W2, b2):
    Wf, csum, T, xsplit, M = _prep(x, W0, b0, initial_embed, Wg, a_src, a_dst)
    zden = jnp.zeros((ROWS_PER_TILE, 16), jnp.float32)
    zmsg = jnp.zeros((ROWS_PER_TILE, MSGW), jnp.float32)
    den, w4 = _sc_denominators(edge_index, T, M, zden)
    acc = _sc_messages(edge_index, xsplit.reshape(2 * N, 64),
                       w4.reshape(E * 4), zmsg)
    samp, logp = _post(acc[0], acc[1], den[0], den[1], x, mask, Wf, csum,
                       W1, b1, W2, b2,
                       graph_id_index.reshape(NBLK, G // NBLK, 1))
    return samp.reshape(G), logp.reshape(G)


# speedup vs baseline: 75.9696x; 1.4281x over previous
import functools

import jax
import jax.numpy as jnp
from jax import lax
from jax.experimental import pallas as pl
from jax.experimental.pallas import tpu as pltpu
from jax.experimental.pallas import tpu_sc as plsc

N = 10000
E = 320000
F = 128
H = 128
HEADS = 3
G = 100
CITIES = N // G

MSGW = 192
CH1 = 80
EPT1 = E // 32
NCH1 = EPT1 // CH1
CH2 = 32
EPT2 = E // 16
NCH2 = EPT2 // CH2
ROWS_PER_TILE = N // 16
NB = 1000
NBLK = N // NB


def _a_body(W0_r, b0_r, ie_r, Wg_r, asrc_r, adst_r, x_r,
            Wf_r, csum_r, T_r, xlo_r, xhi_r, M_r, V_s, cc_s):
    i = pl.program_id(0)

    @pl.when(i == 0)
    def _():
        csum = jnp.zeros((1, 2 * H), jnp.float32)
        vcols = []
        dcols = []
        ccs = []
        ccd = []
        for h in range(HEADS):
            Wg_top = Wg_r[h, :H, :]
            Wg_bot = Wg_r[h, H:, :]
            Wf = jnp.einsum('kf,ko->fo', W0_r[...], Wg_bot,
                            preferred_element_type=jnp.float32)
            Wf_r[h, :, :] = Wf
            const = (jnp.einsum('xk,ko->xo', ie_r[...], Wg_top,
                                preferred_element_type=jnp.float32)
                     + jnp.einsum('xk,ko->xo', b0_r[...], Wg_bot,
                                  preferred_element_type=jnp.float32))
            csum = csum + const
            asr = asrc_r[h:h + 1, :]
            adr = adst_r[h:h + 1, :]
            vcols.append(jnp.einsum('fo,xo->fx', Wf, asr,
                                    preferred_element_type=jnp.float32))
            dcols.append(jnp.einsum('fo,xo->fx', Wf, adr,
                                    preferred_element_type=jnp.float32))
            ccs.append(jnp.einsum('xo,yo->xy', const, asr,
                                  preferred_element_type=jnp.float32))
            ccd.append(jnp.einsum('xo,yo->xy', const, adr,
                                  preferred_element_type=jnp.float32))
        zc = jnp.zeros((F, 2), jnp.float32)
        V_s[...] = jnp.concatenate(vcols + dcols + [zc], axis=1)
        cc_s[...] = jnp.concatenate(ccs + ccd
                                    + [jnp.zeros((1, 2), jnp.float32)], axis=1)
        csum_r[...] = csum

    xv = x_r[...]
    T = jnp.dot(xv, V_s[...], preferred_element_type=jnp.float32) + cc_s[...]
    T_r[...] = T
    xlo_r[...] = xv[:, :64]
    xhi_r[...] = xv[:, 64:]
    bm = jnp.max(T, axis=0, keepdims=True)

    @pl.when(i == 0)
    def _():
        M_r[:, :8] = bm
        M_r[:, 8:] = bm

    @pl.when(i > 0)
    def _():
        M_r[:, :8] = jnp.maximum(M_r[:, :8], bm)

    @pl.when(i == NBLK - 1)
    def _():
        cm = M_r[:, :8]
        r = lax.broadcasted_iota(jnp.int32, (8, 16), 0)
        c = lax.broadcasted_iota(jnp.int32, (8, 16), 1)
        sel = jnp.where((c < 3) & ((r == c) | (r == c + 3)), 1.0, 0.0)
        m = jnp.dot(cm, sel, preferred_element_type=jnp.float32)
        M_r[...] = jnp.where(m > 0, m, 0.2 * m)


def _prep(x, W0, b0, initial_embed, Wg, a_src, a_dst):
    return pl.pallas_call(
        _a_body,
        grid=(NBLK,),
        in_specs=[
            pl.BlockSpec((F, F), lambda i: (0, 0)),
            pl.BlockSpec((1, H), lambda i: (0, 0)),
            pl.BlockSpec((1, H), lambda i: (0, 0)),
            pl.BlockSpec((HEADS, 2 * H, 2 * H), lambda i: (0, 0, 0)),
            pl.BlockSpec((HEADS, 2 * H), lambda i: (0, 0)),
            pl.BlockSpec((HEADS, 2 * H), lambda i: (0, 0)),
            pl.BlockSpec((NB, F), lambda i: (i, 0)),
        ],
        out_specs=(
            pl.BlockSpec((HEADS, F, 2 * H), lambda i: (0, 0, 0)),
            pl.BlockSpec((1, 2 * H), lambda i: (0, 0)),
            pl.BlockSpec((NB, 8), lambda i: (i, 0)),
            pl.BlockSpec((NB, 64), lambda i: (i, 0)),
            pl.BlockSpec((NB, 64), lambda i: (i, 0)),
            pl.BlockSpec((1, 16), lambda i: (0, 0)),
        ),
        out_shape=(
            jax.ShapeDtypeStruct((HEADS, F, 2 * H), jnp.float32),
            jax.ShapeDtypeStruct((1, 2 * H), jnp.float32),
            jax.ShapeDtypeStruct((N, 8), jnp.float32),
            jax.ShapeDtypeStruct((N, 64), jnp.float32),
            jax.ShapeDtypeStruct((N, 64), jnp.float32),
            jax.ShapeDtypeStruct((1, 16), jnp.float32),
        ),
        scratch_shapes=[
            pltpu.VMEM((F, 8), jnp.float32),
            pltpu.VMEM((1, 8), jnp.float32),
        ],
    )(W0, b0.reshape(1, H), initial_embed.reshape(1, H), Wg, a_src, a_dst, x)


_SC_PARAMS = pltpu.CompilerParams(use_tc_tiling_on_sc=False,
                                  needs_layout_passes=False)
_LANE = lambda: lax.broadcasted_iota(jnp.int32, (16,), 0)


def _den_body(ei_hbm, T_hbm, M_hbm, z_hbm,
              out_hbm, w4_hbm,
              src_v, dst_v,
              srr0, srr1, dsc0, dsc1, dsp0, dsp1,
              tsr0, tsr1, tdr0, tdr1,
              wst0, wst1, w4st0, w4st1, M_v,
              acc_sh,
              sin, st0, st1, ss, sw):
    cid = lax.axis_index("c")
    sid = lax.axis_index("s")
    pltpu.sync_copy(M_hbm, M_v)
    pltpu.sync_copy(z_hbm,
                    acc_sh.at[pl.ds(sid * ROWS_PER_TILE, ROWS_PER_TILE)])
    zero16 = jnp.zeros((16,), jnp.float32)
    for r in range(CH1):
        wst0[r, pl.ds(0, 16)] = zero16
        wst1[r, pl.ds(0, 16)] = zero16
    plsc.subcore_barrier()
    mv = M_v[0, :]
    m0, m1, m2 = mv[0], mv[1], mv[2]

    srrs = (srr0, srr1)
    dscs = (dsc0, dsc1)
    dsps = (dsp0, dsp1)
    tsrs = (tsr0, tsr1)
    tdrs = (tdr0, tdr1)
    wsts = (wst0, wst1)
    w4sts = (w4st0, w4st1)
    sts = (st0, st1)

    def issue_in(j):
        cj = jnp.minimum(j, NCH1 - 1)
        base = (cid * 16 + sid) * EPT1 + cj * CH1
        pltpu.async_copy(ei_hbm.at[0, pl.ds(base, CH1)], src_v, sin)
        pltpu.async_copy(ei_hbm.at[1, pl.ds(base, CH1)], dst_v, sin)

    def wait_in():
        pltpu.make_async_copy(ei_hbm.at[0, pl.ds(0, CH1)], src_v, sin).wait()
        pltpu.make_async_copy(ei_hbm.at[1, pl.ds(0, CH1)], dst_v, sin).wait()

    def extract(b):
        for g in range(CH1 // 16):
            sl = pl.ds(g * 16, 16)
            srrs[b][sl] = src_v[sl]
            dscs[b][sl] = dst_v[sl]

    def issue_trows(b):
        pltpu.async_copy(T_hbm.at[srrs[b]], tsrs[b], sts[b])
        pltpu.async_copy(T_hbm.at[dscs[b]], tdrs[b], sts[b])

    def wait_trows(b):
        pltpu.make_async_copy(T_hbm.at[srrs[b]], tsrs[b], sts[b]).wait()
        pltpu.make_async_copy(T_hbm.at[dscs[b]], tdrs[b], sts[b]).wait()

    def compute(b):
        for g in range(CH1 // 16):
            grows = jnp.full((16,), g * 16, jnp.int32) + _LANE()
            c0 = jnp.full((16,), 0, jnp.int32)
            for h, m in ((0, m0), (1, m1), (2, m2)):
                es = plsc.load_gather(tsrs[b], [grows, c0 + h])
                ed = plsc.load_gather(tdrs[b], [grows, c0 + (3 + h)])
                ev = es + ed
                ev = jnp.where(ev < 0, 0.2 * ev, ev)
                w = jnp.exp(ev - jnp.full((16,), m, jnp.float32))
                plsc.store_scatter(wsts[b], [grows, c0 + h], w)
                plsc.store_scatter(w4sts[b], [grows, c0 + h], w)

    def issue_scat(j, b):
        for g in range(CH1 // 16):
            sl = pl.ds(g * 16, 16)
            dsps[b][sl] = dscs[b][sl]
        pltpu.async_copy(wsts[b], acc_sh.at[dsps[b]], ss, add=True)
        cj = jnp.minimum(j, NCH1 - 1)
        base = (cid * 16 + sid) * EPT1 + cj * CH1
        pltpu.async_copy(w4sts[b], w4_hbm.at[pl.ds(base, CH1)], sw)

    def wait_scat(b):
        pltpu.make_async_copy(wsts[b], acc_sh.at[dsps[b]], ss).wait()
        pltpu.make_async_copy(w4sts[b], w4_hbm.at[pl.ds(0, CH1)], sw).wait()

    def body(j, b, do_wait_scat):
        nb = 1 - b
        wait_in()
        extract(nb)
        issue_in(j + 2)
        issue_trows(nb)
        wait_trows(b)
        if do_wait_scat:
            wait_scat(b)
        compute(b)
        issue_scat(j, b)

    issue_in(0)
    wait_in()
    extract(0)
    issue_trows(0)
    issue_in(1)

    body(0, 0, False)
    body(1, 1, False)
    body(2, 0, True)

    def two(j):
        body(j, 1, True)
        body(j + 1, 0, True)

    pl.loop(3, NCH1, step=2)(two)

    wait_scat((NCH1 - 1) % 2)
    wait_scat(NCH1 % 2)
    wait_trows(NCH1 % 2)
    wait_in()

    plsc.subcore_barrier()
    pltpu.sync_copy(
        acc_sh.at[pl.ds(sid * ROWS_PER_TILE, ROWS_PER_TILE)],
        out_hbm.at[cid, pl.ds(sid * ROWS_PER_TILE, ROWS_PER_TILE)])


def _sc_denominators(edge_index, T, M, zrows16):
    mesh = plsc.VectorSubcoreMesh(core_axis_name="c", subcore_axis_name="s")
    f = pl.kernel(
        _den_body,
        out_type=(jax.ShapeDtypeStruct((2, N, 16), jnp.float32),
                  jax.ShapeDtypeStruct((E, 4), jnp.float32)),
        mesh=mesh,
        compiler_params=_SC_PARAMS,
        scratch_types=[
            pltpu.VMEM((CH1,), jnp.int32),
            pltpu.VMEM((CH1,), jnp.int32),
            pltpu.VMEM((CH1,), jnp.int32),
            pltpu.VMEM((CH1,), jnp.int32),
            pltpu.VMEM((CH1,), jnp.int32),
            pltpu.VMEM((CH1,), jnp.int32),
            pltpu.VMEM((CH1,), jnp.int32),
            pltpu.VMEM((CH1,), jnp.int32),
            pltpu.VMEM((CH1, 8), jnp.float32),
            pltpu.VMEM((CH1, 8), jnp.float32),
            pltpu.VMEM((CH1, 8), jnp.float32),
            pltpu.VMEM((CH1, 8), jnp.float32),
            pltpu.VMEM((CH1, 16), jnp.float32),
            pltpu.VMEM((CH1, 16), jnp.float32),
            pltpu.VMEM((CH1, 4), jnp.float32),
            pltpu.VMEM((CH1, 4), jnp.float32),
            pltpu.VMEM((1, 16), jnp.float32),
            pltpu.VMEM_SHARED((N, 16), jnp.float32),
            pltpu.SemaphoreType.DMA,
            pltpu.SemaphoreType.DMA,
            pltpu.SemaphoreType.DMA,
            pltpu.SemaphoreType.DMA,
            pltpu.SemaphoreType.DMA,
        ],
    )
    return f(edge_index, T, M, zrows16)


def _msg_body(ei_hbm, xlo_hbm, xhi_hbm, w4_hbm, z_hbm,
              out_hbm,
              src_v, dst_v,
              idx0, idx1, dsc0, dsc1, dsp0, dsp1, w4c0, w4c1,
              rows0, rows1, stag_v,
              acc_sh,
              sin, sr0, sr1, ss):
    cid = lax.axis_index("c")
    sid = lax.axis_index("s")
    pltpu.sync_copy(z_hbm,
                    acc_sh.at[pl.ds(sid * ROWS_PER_TILE, ROWS_PER_TILE)])
    plsc.subcore_barrier()

    idxs = (idx0, idx1)
    dscs = (dsc0, dsc1)
    dsps = (dsp0, dsp1)
    w4cs = (w4c0, w4c1)
    rowss = (rows0, rows1)
    srs = (sr0, sr1)

    def issue_in(j):
        cj = jnp.minimum(j, NCH2 - 1)
        base = sid * EPT2 + cj * CH2
        pltpu.async_copy(ei_hbm.at[0, pl.ds(base, CH2)], src_v, sin)
        pltpu.async_copy(ei_hbm.at[1, pl.ds(base, CH2)], dst_v, sin)

    def wait_in():
        pltpu.make_async_copy(ei_hbm.at[0, pl.ds(0, CH2)], src_v, sin).wait()
        pltpu.make_async_copy(ei_hbm.at[1, pl.ds(0, CH2)], dst_v, sin).wait()

    def extract(b):
        for g in range(CH2 // 16):
            sl = pl.ds(g * 16, 16)
            idxs[b][sl] = src_v[sl]
            dscs[b][sl] = dst_v[sl]

    def issue_rows(j, b):
        cj = jnp.minimum(j, NCH2 - 1)
        base = sid * EPT2 + cj * CH2
        pltpu.async_copy(w4_hbm.at[pl.ds(base, CH2)], w4cs[b], srs[b])

        @pl.when(cid == 0)
        def _():
            pltpu.async_copy(xlo_hbm.at[idxs[b]], rowss[b], srs[b])

        @pl.when(cid == 1)
        def _():
            pltpu.async_copy(xhi_hbm.at[idxs[b]], rowss[b], srs[b])

    def wait_rows(b):
        pltpu.make_async_copy(w4_hbm.at[pl.ds(0, CH2)], w4cs[b],
                              srs[b]).wait()

        @pl.when(cid == 0)
        def _():
            pltpu.make_async_copy(xlo_hbm.at[idxs[b]], rowss[b],
                                  srs[b]).wait()

        @pl.when(cid == 1)
        def _():
            pltpu.make_async_copy(xhi_hbm.at[idxs[b]], rowss[b],
                                  srs[b]).wait()

    def issue_scat(b):
        for g in range(CH2 // 16):
            sl = pl.ds(g * 16, 16)
            dsps[b][sl] = dscs[b][sl]
        pltpu.async_copy(stag_v, acc_sh.at[dsps[b]], ss, add=True)

    def wait_scat(b):
        pltpu.make_async_copy(stag_v, acc_sh.at[dsps[b]], ss).wait()

    def compute(b):
        w4c = w4cs[b]
        rows = rowss[b]
        for g in range(CH2 // 16):
            grows = jnp.full((16,), g * 16, jnp.int32) + _LANE()
            c0 = jnp.full((16,), 0, jnp.int32)
            wv0 = plsc.load_gather(w4c, [grows, c0])
            wv1 = plsc.load_gather(w4c, [grows, c0 + 1])
            wv2 = plsc.load_gather(w4c, [grows, c0 + 2])
            for i in range(16):
                e = g * 16 + i
                wb0 = jnp.full((16,), wv0[i], jnp.float32)
                wb1 = jnp.full((16,), wv1[i], jnp.float32)
                wb2 = jnp.full((16,), wv2[i], jnp.float32)
                for cb in range(4):
                    xvv = rows[e, pl.ds(cb * 16, 16)]
                    stag_v[e, pl.ds(cb * 16, 16)] = xvv * wb0
                    stag_v[e, pl.ds(64 + cb * 16, 16)] = xvv * wb1
                    stag_v[e, pl.ds(128 + cb * 16, 16)] = xvv * wb2

    def body(j, b, first):
        nb = 1 - b
        wait_in()
        extract(nb)
        issue_in(j + 2)
        issue_rows(j + 1, nb)
        wait_rows(b)
        if first:
            @pl.when(j > 0)
            def _():
                wait_scat(nb)
        else:
            wait_scat(nb)
        compute(b)
        issue_scat(b)

    issue_in(0)
    wait_in()
    extract(0)
    issue_rows(0, 0)
    issue_in(1)

    def two(j):
        body(j, 0, True)
        body(j + 1, 1, False)

    pl.loop(0, NCH2 - 1, step=2)(two)
    body(NCH2 - 1, (NCH2 - 1) % 2, False)

    wait_scat((NCH2 - 1) % 2)
    wait_rows(NCH2 % 2)
    wait_in()

    plsc.subcore_barrier()
    pltpu.sync_copy(
        acc_sh.at[pl.ds(sid * ROWS_PER_TILE, ROWS_PER_TILE)],
        out_hbm.at[cid, pl.ds(sid * ROWS_PER_TILE, ROWS_PER_TILE)])


def _sc_messages(edge_index, xlo, xhi, w4, zrows):
    mesh = plsc.VectorSubcoreMesh(core_axis_name="c", subcore_axis_name="s")
    f = pl.kernel(
        _msg_body,
        out_type=jax.ShapeDtypeStruct((2, N, MSGW), jnp.float32),
        mesh=mesh,
        compiler_params=_SC_PARAMS,
        scratch_types=[
            pltpu.VMEM((CH2,), jnp.int32),
            pltpu.VMEM((CH2,), jnp.int32),
            pltpu.VMEM((CH2,), jnp.int32),
            pltpu.VMEM((CH2,), jnp.int32),
            pltpu.VMEM((CH2,), jnp.int32),
            pltpu.VMEM((CH2,), jnp.int32),
            pltpu.VMEM((CH2,), jnp.int32),
            pltpu.VMEM((CH2,), jnp.int32),
            pltpu.VMEM((CH2, 4), jnp.float32),
            pltpu.VMEM((CH2, 4), jnp.float32),
            pltpu.VMEM((CH2, 64), jnp.float32),
            pltpu.VMEM((CH2, 64), jnp.float32),
            pltpu.VMEM((CH2, MSGW), jnp.float32),
            pltpu.VMEM_SHARED((N, MSGW), jnp.float32),
            pltpu.SemaphoreType.DMA,
            pltpu.SemaphoreType.DMA,
            pltpu.SemaphoreType.DMA,
            pltpu.SemaphoreType.DMA,
        ],
    )
    return f(edge_index, xlo, xhi, w4, zrows)


def _post_body(a0_r, a1_r, d0_r, d1_r, x_r, mask_r, Wf_r, csum_r, W1_r, b1_r,
               W2_r, b2_r, gid_r, samp_r, logp_r):
    den3 = d0_r[...] + d1_r[...]
    has = den3[:, 0:1] > 0
    osum = jnp.where(has, csum_r[...], 0.0)
    for h in range(HEADS):
        den = den3[:, h:h + 1]
        agg = jnp.concatenate(
            [a0_r[:, h * 64:(h + 1) * 64], a1_r[:, h * 64:(h + 1) * 64]],
            axis=1)
        agg = jnp.where(den > 0, agg / jnp.where(den > 0, den, 1.0), 0.0)
        osum = osum + jnp.dot(agg, Wf_r[h, :, :],
                              preferred_element_type=jnp.float32)
    enc = osum * (1.0 / 3.0)
    enc = jnp.where(enc > 0, enc, jnp.exp(jnp.minimum(enc, 0.0)) - 1.0)
    encg = enc.reshape(NB // CITIES, CITIES, 2 * H)
    center = jnp.broadcast_to(encg[:, 0:1, :], encg.shape)
    crep = center.reshape(NB, 2 * H)
    h1 = (jnp.einsum('nc,kc->nk', crep, W1_r[:, :2 * H],
                     preferred_element_type=jnp.float32)
          + jnp.einsum('nc,kc->nk', enc, W1_r[:, 2 * H:],
                       preferred_element_type=jnp.float32)
          + b1_r[...])
    h1 = jnp.maximum(h1, 0.0)
    lg = (jnp.einsum('nf,of->no', x_r[...], W2_r[:, :F],
                     preferred_element_type=jnp.float32)
          + jnp.einsum('nh,oh->no', h1, W2_r[:, F:],
                       preferred_element_type=jnp.float32)
          + b2_r[...])
    lg = lg - jnp.where(mask_r[...], 0.0, 1e6)
    iota = lax.broadcasted_iota(jnp.int32, (CITIES, 1), 0)
    for g in range(NB // CITIES):
        l = lg[g * CITIES:(g + 1) * CITIES, :]
        m = jnp.max(l, axis=0, keepdims=True)
        den = jnp.sum(jnp.exp(l - m), axis=0, keepdims=True)
        cand = jnp.where(l >= m, iota, N)
        samp = jnp.min(cand, axis=0, keepdims=True)
        samp_r[0, pl.ds(g, 1), :] = samp + gid_r[0, pl.ds(g, 1), :]
        logp_r[0, pl.ds(g, 1), :] = -jnp.log(den)


def _post(a0, a1, d0, d1, x, mask, Wf, csum, W1, b1, W2, b2, gid2d):
    return pl.pallas_call(
        _post_body,
        grid=(NBLK,),
        in_specs=[
            pl.BlockSpec((NB, MSGW), lambda i: (i, 0)),
            pl.BlockSpec((NB, MSGW), lambda i: (i, 0)),
            pl.BlockSpec((NB, 16), lambda i: (i, 0)),
            pl.BlockSpec((NB, 16), lambda i: (i, 0)),
            pl.BlockSpec((NB, F), lambda i: (i, 0)),
            pl.BlockSpec((NB, 1), lambda i: (i, 0)),
            pl.BlockSpec((HEADS, F, 2 * H), lambda i: (0, 0, 0)),
            pl.BlockSpec((1, 2 * H), lambda i: (0, 0)),
            pl.BlockSpec((H, 4 * H), lambda i: (0, 0)),
            pl.BlockSpec((1, H), lambda i: (0, 0)),
            pl.BlockSpec((1, H + F), lambda i: (0, 0)),
            pl.BlockSpec((1, 1), lambda i: (0, 0)),
            pl.BlockSpec((1, G // NBLK, 1), lambda i: (i, 0, 0)),
        ],
        out_specs=(
            pl.BlockSpec((1, G // NBLK, 1), lambda i: (i, 0, 0)),
            pl.BlockSpec((1, G // NBLK, 1), lambda i: (i, 0, 0)),
        ),
        out_shape=(
            jax.ShapeDtypeStruct((NBLK, G // NBLK, 1), jnp.int32),
            jax.ShapeDtypeStruct((NBLK, G // NBLK, 1), jnp.float32),
        ),
    )(a0, a1, d0, d1, x, mask, Wf, csum, W1, b1.reshape(1, H), W2,
      b2.reshape(1, 1), gid2d)


def kernel(x, edge_index, batch, center_node_index, mask, graph_id_index,
           W0, b0, initial_embed, Wg, a_src, a_dst, W1, b1, W2, b2):
    Wf, csum, T, xlo, xhi, M = _prep(x, W0, b0, initial_embed, Wg,
                                     a_src, a_dst)
    zden = jnp.zeros((ROWS_PER_TILE, 16), jnp.float32)
    zmsg = jnp.zeros((ROWS_PER_TILE, MSGW), jnp.float32)
    den, w4 = _sc_denominators(edge_index, T, M, zden)
    acc = _sc_messages(edge_index, xlo, xhi, w4, zmsg)
    samp, logp = _post(acc[0], acc[1], den[0], den[1], x, mask, Wf, csum,
                       W1, b1, W2, b2,
                       graph_id_index.reshape(NBLK, G // NBLK, 1))
    return samp.reshape(G), logp.reshape(G)


# speedup vs baseline: 81.7809x; 1.0765x over previous
import functools

import jax
import jax.numpy as jnp
from jax import lax
from jax.experimental import pallas as pl
from jax.experimental.pallas import tpu as pltpu
from jax.experimental.pallas import tpu_sc as plsc

N = 10000
E = 320000
F = 128
H = 128
HEADS = 3
G = 100
CITIES = N // G

MSGW = 192
CH1 = 80
EPT1 = E // 32
NCH1 = EPT1 // CH1
CH2 = 32
EPT2 = E // 16
NCH2 = EPT2 // CH2
ROWS_PER_TILE = N // 16
NB = 1000
NBLK = N // NB


def _a_body(W0_r, b0_r, ie_r, Wg_r, asrc_r, adst_r, x_r,
            Wf_r, csum_r, T_r, xlo_r, xhi_r, M_r, V_s, cc_s):
    i = pl.program_id(0)

    @pl.when(i == 0)
    def _():
        csum = jnp.zeros((1, 2 * H), jnp.float32)
        vcols = []
        dcols = []
        ccs = []
        ccd = []
        for h in range(HEADS):
            Wg_top = Wg_r[h, :H, :]
            Wg_bot = Wg_r[h, H:, :]
            Wf = jnp.einsum('kf,ko->fo', W0_r[...], Wg_bot,
                            preferred_element_type=jnp.float32)
            Wf_r[h, :, :] = Wf
            const = (jnp.einsum('xk,ko->xo', ie_r[...], Wg_top,
                                preferred_element_type=jnp.float32)
                     + jnp.einsum('xk,ko->xo', b0_r[...], Wg_bot,
                                  preferred_element_type=jnp.float32))
            csum = csum + const
            asr = asrc_r[h:h + 1, :]
            adr = adst_r[h:h + 1, :]
            vcols.append(jnp.einsum('fo,xo->fx', Wf, asr,
                                    preferred_element_type=jnp.float32))
            dcols.append(jnp.einsum('fo,xo->fx', Wf, adr,
                                    preferred_element_type=jnp.float32))
            ccs.append(jnp.einsum('xo,yo->xy', const, asr,
                                  preferred_element_type=jnp.float32))
            ccd.append(jnp.einsum('xo,yo->xy', const, adr,
                                  preferred_element_type=jnp.float32))
        zc = jnp.zeros((F, 2), jnp.float32)
        V_s[...] = jnp.concatenate(vcols + dcols + [zc], axis=1)
        cc_s[...] = jnp.concatenate(ccs + ccd
                                    + [jnp.zeros((1, 2), jnp.float32)], axis=1)
        csum_r[...] = csum

    xv = x_r[...]
    T = jnp.dot(xv, V_s[...], preferred_element_type=jnp.float32) + cc_s[...]
    T_r[...] = T
    xlo_r[...] = xv[:, :64]
    xhi_r[...] = xv[:, 64:]
    bm = jnp.max(T, axis=0, keepdims=True)

    @pl.when(i == 0)
    def _():
        M_r[:, :8] = bm
        M_r[:, 8:] = bm

    @pl.when(i > 0)
    def _():
        M_r[:, :8] = jnp.maximum(M_r[:, :8], bm)

    @pl.when(i == NBLK - 1)
    def _():
        cm = M_r[:, :8]
        r = lax.broadcasted_iota(jnp.int32, (8, 16), 0)
        c = lax.broadcasted_iota(jnp.int32, (8, 16), 1)
        sel = jnp.where((c < 3) & ((r == c) | (r == c + 3)), 1.0, 0.0)
        m = jnp.dot(cm, sel, preferred_element_type=jnp.float32)
        M_r[...] = jnp.where(m > 0, m, 0.2 * m)


def _prep(x, W0, b0, initial_embed, Wg, a_src, a_dst):
    return pl.pallas_call(
        _a_body,
        grid=(NBLK,),
        in_specs=[
            pl.BlockSpec((F, F), lambda i: (0, 0)),
            pl.BlockSpec((1, H), lambda i: (0, 0)),
            pl.BlockSpec((1, H), lambda i: (0, 0)),
            pl.BlockSpec((HEADS, 2 * H, 2 * H), lambda i: (0, 0, 0)),
            pl.BlockSpec((HEADS, 2 * H), lambda i: (0, 0)),
            pl.BlockSpec((HEADS, 2 * H), lambda i: (0, 0)),
            pl.BlockSpec((NB, F), lambda i: (i, 0)),
        ],
        out_specs=(
            pl.BlockSpec((HEADS, F, 2 * H), lambda i: (0, 0, 0)),
            pl.BlockSpec((1, 2 * H), lambda i: (0, 0)),
            pl.BlockSpec((NB, 8), lambda i: (i, 0)),
            pl.BlockSpec((NB, 64), lambda i: (i, 0)),
            pl.BlockSpec((NB, 64), lambda i: (i, 0)),
            pl.BlockSpec((1, 16), lambda i: (0, 0)),
        ),
        out_shape=(
            jax.ShapeDtypeStruct((HEADS, F, 2 * H), jnp.float32),
            jax.ShapeDtypeStruct((1, 2 * H), jnp.float32),
            jax.ShapeDtypeStruct((N, 8), jnp.float32),
            jax.ShapeDtypeStruct((N, 64), jnp.float32),
            jax.ShapeDtypeStruct((N, 64), jnp.float32),
            jax.ShapeDtypeStruct((1, 16), jnp.float32),
        ),
        scratch_shapes=[
            pltpu.VMEM((F, 8), jnp.float32),
            pltpu.VMEM((1, 8), jnp.float32),
        ],
    )(W0, b0.reshape(1, H), initial_embed.reshape(1, H), Wg, a_src, a_dst, x)


_SC_PARAMS = pltpu.CompilerParams(use_tc_tiling_on_sc=False,
                                  needs_layout_passes=False)
_LANE = lambda: lax.broadcasted_iota(jnp.int32, (16,), 0)


def _den_body(ei_hbm, T_hbm, M_hbm, z_hbm,
              out_hbm, w4_hbm,
              src_v, dst_v,
              srr0, srr1, dsc0, dsc1, dsp0, dsp1,
              tsr0, tsr1, tdr0, tdr1,
              wst0, wst1, w4st0, w4st1, M_v,
              acc_sh, T_sh,
              sin, st0, st1, ss, sw):
    cid = lax.axis_index("c")
    sid = lax.axis_index("s")
    pltpu.sync_copy(M_hbm, M_v)
    pltpu.sync_copy(z_hbm,
                    acc_sh.at[pl.ds(sid * ROWS_PER_TILE, ROWS_PER_TILE)])
    pltpu.sync_copy(T_hbm.at[pl.ds(sid * ROWS_PER_TILE, ROWS_PER_TILE)],
                    T_sh.at[pl.ds(sid * ROWS_PER_TILE, ROWS_PER_TILE)])
    zero16 = jnp.zeros((16,), jnp.float32)
    for r in range(CH1):
        wst0[r, pl.ds(0, 16)] = zero16
        wst1[r, pl.ds(0, 16)] = zero16
    plsc.subcore_barrier()
    mv = M_v[0, :]
    m0, m1, m2 = mv[0], mv[1], mv[2]

    srrs = (srr0, srr1)
    dscs = (dsc0, dsc1)
    dsps = (dsp0, dsp1)
    tsrs = (tsr0, tsr1)
    tdrs = (tdr0, tdr1)
    wsts = (wst0, wst1)
    w4sts = (w4st0, w4st1)
    sts = (st0, st1)

    def issue_in(j):
        cj = jnp.minimum(j, NCH1 - 1)
        base = (cid * 16 + sid) * EPT1 + cj * CH1
        pltpu.async_copy(ei_hbm.at[0, pl.ds(base, CH1)], src_v, sin)
        pltpu.async_copy(ei_hbm.at[1, pl.ds(base, CH1)], dst_v, sin)

    def wait_in():
        pltpu.make_async_copy(ei_hbm.at[0, pl.ds(0, CH1)], src_v, sin).wait()
        pltpu.make_async_copy(ei_hbm.at[1, pl.ds(0, CH1)], dst_v, sin).wait()

    def extract(b):
        for g in range(CH1 // 16):
            sl = pl.ds(g * 16, 16)
            srrs[b][sl] = src_v[sl]
            dscs[b][sl] = dst_v[sl]

    def issue_trows(b):
        pltpu.async_copy(T_sh.at[srrs[b]], tsrs[b], sts[b])
        pltpu.async_copy(T_sh.at[dscs[b]], tdrs[b], sts[b])

    def wait_trows(b):
        pltpu.make_async_copy(T_sh.at[srrs[b]], tsrs[b], sts[b]).wait()
        pltpu.make_async_copy(T_sh.at[dscs[b]], tdrs[b], sts[b]).wait()

    def compute(b):
        for g in range(CH1 // 16):
            grows = jnp.full((16,), g * 16, jnp.int32) + _LANE()
            c0 = jnp.full((16,), 0, jnp.int32)
            for h, m in ((0, m0), (1, m1), (2, m2)):
                es = plsc.load_gather(tsrs[b], [grows, c0 + h])
                ed = plsc.load_gather(tdrs[b], [grows, c0 + (3 + h)])
                ev = es + ed
                ev = jnp.where(ev < 0, 0.2 * ev, ev)
                w = jnp.exp(ev - jnp.full((16,), m, jnp.float32))
                plsc.store_scatter(wsts[b], [grows, c0 + h], w)
                plsc.store_scatter(w4sts[b], [grows, c0 + h], w)

    def issue_scat(j, b):
        for g in range(CH1 // 16):
            sl = pl.ds(g * 16, 16)
            dsps[b][sl] = dscs[b][sl]
        pltpu.async_copy(wsts[b], acc_sh.at[dsps[b]], ss, add=True)
        cj = jnp.minimum(j, NCH1 - 1)
        base = (cid * 16 + sid) * EPT1 + cj * CH1
        pltpu.async_copy(w4sts[b], w4_hbm.at[pl.ds(base, CH1)], sw)

    def wait_scat(b):
        pltpu.make_async_copy(wsts[b], acc_sh.at[dsps[b]], ss).wait()
        pltpu.make_async_copy(w4sts[b], w4_hbm.at[pl.ds(0, CH1)], sw).wait()

    def body(j, b, do_wait_scat):
        nb = 1 - b
        wait_in()
        extract(nb)
        issue_in(j + 2)
        issue_trows(nb)
        wait_trows(b)
        if do_wait_scat:
            wait_scat(b)
        compute(b)
        issue_scat(j, b)

    issue_in(0)
    wait_in()
    extract(0)
    issue_trows(0)
    issue_in(1)

    body(0, 0, False)
    body(1, 1, False)
    body(2, 0, True)

    def two(j):
        body(j, 1, True)
        body(j + 1, 0, True)

    pl.loop(3, NCH1, step=2)(two)

    wait_scat((NCH1 - 1) % 2)
    wait_scat(NCH1 % 2)
    wait_trows(NCH1 % 2)
    wait_in()

    plsc.subcore_barrier()
    pltpu.sync_copy(
        acc_sh.at[pl.ds(sid * ROWS_PER_TILE, ROWS_PER_TILE)],
        out_hbm.at[cid, pl.ds(sid * ROWS_PER_TILE, ROWS_PER_TILE)])


def _sc_denominators(edge_index, T, M, zrows16):
    mesh = plsc.VectorSubcoreMesh(core_axis_name="c", subcore_axis_name="s")
    f = pl.kernel(
        _den_body,
        out_type=(jax.ShapeDtypeStruct((2, N, 16), jnp.float32),
                  jax.ShapeDtypeStruct((E, 4), jnp.float32)),
        mesh=mesh,
        compiler_params=_SC_PARAMS,
        scratch_types=[
            pltpu.VMEM((CH1,), jnp.int32),
            pltpu.VMEM((CH1,), jnp.int32),
            pltpu.VMEM((CH1,), jnp.int32),
            pltpu.VMEM((CH1,), jnp.int32),
            pltpu.VMEM((CH1,), jnp.int32),
            pltpu.VMEM((CH1,), jnp.int32),
            pltpu.VMEM((CH1,), jnp.int32),
            pltpu.VMEM((CH1,), jnp.int32),
            pltpu.VMEM((CH1, 8), jnp.float32),
            pltpu.VMEM((CH1, 8), jnp.float32),
            pltpu.VMEM((CH1, 8), jnp.float32),
            pltpu.VMEM((CH1, 8), jnp.float32),
            pltpu.VMEM((CH1, 16), jnp.float32),
            pltpu.VMEM((CH1, 16), jnp.float32),
            pltpu.VMEM((CH1, 4), jnp.float32),
            pltpu.VMEM((CH1, 4), jnp.float32),
            pltpu.VMEM((1, 16), jnp.float32),
            pltpu.VMEM_SHARED((N, 16), jnp.float32),
            pltpu.VMEM_SHARED((N, 8), jnp.float32),
            pltpu.SemaphoreType.DMA,
            pltpu.SemaphoreType.DMA,
            pltpu.SemaphoreType.DMA,
            pltpu.SemaphoreType.DMA,
            pltpu.SemaphoreType.DMA,
        ],
    )
    return f(edge_index, T, M, zrows16)


def _msg_body(ei_hbm, xlo_hbm, xhi_hbm, w4_hbm, z_hbm,
              out_hbm,
              src_v, dst_v,
              idx0, idx1, dsc0, dsc1, dsp0, dsp1, w4c0, w4c1,
              rows0, rows1, stag_v,
              acc_sh,
              sin, sr0, sr1, ss):
    cid = lax.axis_index("c")
    sid = lax.axis_index("s")
    pltpu.sync_copy(z_hbm,
                    acc_sh.at[pl.ds(sid * ROWS_PER_TILE, ROWS_PER_TILE)])
    plsc.subcore_barrier()

    idxs = (idx0, idx1)
    dscs = (dsc0, dsc1)
    dsps = (dsp0, dsp1)
    w4cs = (w4c0, w4c1)
    rowss = (rows0, rows1)
    srs = (sr0, sr1)

    def issue_in(j):
        cj = jnp.minimum(j, NCH2 - 1)
        base = sid * EPT2 + cj * CH2
        pltpu.async_copy(ei_hbm.at[0, pl.ds(base, CH2)], src_v, sin)
        pltpu.async_copy(ei_hbm.at[1, pl.ds(base, CH2)], dst_v, sin)

    def wait_in():
        pltpu.make_async_copy(ei_hbm.at[0, pl.ds(0, CH2)], src_v, sin).wait()
        pltpu.make_async_copy(ei_hbm.at[1, pl.ds(0, CH2)], dst_v, sin).wait()

    def extract(b):
        for g in range(CH2 // 16):
            sl = pl.ds(g * 16, 16)
            idxs[b][sl] = src_v[sl]
            dscs[b][sl] = dst_v[sl]

    def issue_rows(j, b):
        cj = jnp.minimum(j, NCH2 - 1)
        base = sid * EPT2 + cj * CH2
        pltpu.async_copy(w4_hbm.at[pl.ds(base, CH2)], w4cs[b], srs[b])

        @pl.when(cid == 0)
        def _():
            pltpu.async_copy(xlo_hbm.at[idxs[b]], rowss[b], srs[b])

        @pl.when(cid == 1)
        def _():
            pltpu.async_copy(xhi_hbm.at[idxs[b]], rowss[b], srs[b])

    def wait_rows(b):
        pltpu.make_async_copy(w4_hbm.at[pl.ds(0, CH2)], w4cs[b],
                              srs[b]).wait()

        @pl.when(cid == 0)
        def _():
            pltpu.make_async_copy(xlo_hbm.at[idxs[b]], rowss[b],
                                  srs[b]).wait()

        @pl.when(cid == 1)
        def _():
            pltpu.make_async_copy(xhi_hbm.at[idxs[b]], rowss[b],
                                  srs[b]).wait()

    def issue_scat(b):
        for g in range(CH2 // 16):
            sl = pl.ds(g * 16, 16)
            dsps[b][sl] = dscs[b][sl]
        pltpu.async_copy(stag_v, acc_sh.at[dsps[b]], ss, add=True)

    def wait_scat(b):
        pltpu.make_async_copy(stag_v, acc_sh.at[dsps[b]], ss).wait()

    def compute(b):
        w4c = w4cs[b]
        rows = rowss[b]
        for g in range(CH2 // 16):
            grows = jnp.full((16,), g * 16, jnp.int32) + _LANE()
            c0 = jnp.full((16,), 0, jnp.int32)
            wv0 = plsc.load_gather(w4c, [grows, c0])
            wv1 = plsc.load_gather(w4c, [grows, c0 + 1])
            wv2 = plsc.load_gather(w4c, [grows, c0 + 2])
            for i in range(16):
                e = g * 16 + i
                wb0 = jnp.full((16,), wv0[i], jnp.float32)
                wb1 = jnp.full((16,), wv1[i], jnp.float32)
                wb2 = jnp.full((16,), wv2[i], jnp.float32)
                for cb in range(4):
                    xvv = rows[e, pl.ds(cb * 16, 16)]
                    stag_v[e, pl.ds(cb * 16, 16)] = xvv * wb0
                    stag_v[e, pl.ds(64 + cb * 16, 16)] = xvv * wb1
                    stag_v[e, pl.ds(128 + cb * 16, 16)] = xvv * wb2

    def body(j, b, first):
        nb = 1 - b
        wait_in()
        extract(nb)
        issue_in(j + 2)
        issue_rows(j + 1, nb)
        wait_rows(b)
        if first:
            @pl.when(j > 0)
            def _():
                wait_scat(nb)
        else:
            wait_scat(nb)
        compute(b)
        issue_scat(b)

    issue_in(0)
    wait_in()
    extract(0)
    issue_rows(0, 0)
    issue_in(1)

    def two(j):
        body(j, 0, True)
        body(j + 1, 1, False)

    pl.loop(0, NCH2 - 1, step=2)(two)
    body(NCH2 - 1, (NCH2 - 1) % 2, False)

    wait_scat((NCH2 - 1) % 2)
    wait_rows(NCH2 % 2)
    wait_in()

    plsc.subcore_barrier()
    pltpu.sync_copy(
        acc_sh.at[pl.ds(sid * ROWS_PER_TILE, ROWS_PER_TILE)],
        out_hbm.at[cid, pl.ds(sid * ROWS_PER_TILE, ROWS_PER_TILE)])


def _sc_messages(edge_index, xlo, xhi, w4, zrows):
    mesh = plsc.VectorSubcoreMesh(core_axis_name="c", subcore_axis_name="s")
    f = pl.kernel(
        _msg_body,
        out_type=jax.ShapeDtypeStruct((2, N, MSGW), jnp.float32),
        mesh=mesh,
        compiler_params=_SC_PARAMS,
        scratch_types=[
            pltpu.VMEM((CH2,), jnp.int32),
            pltpu.VMEM((CH2,), jnp.int32),
            pltpu.VMEM((CH2,), jnp.int32),
            pltpu.VMEM((CH2,), jnp.int32),
            pltpu.VMEM((CH2,), jnp.int32),
            pltpu.VMEM((CH2,), jnp.int32),
            pltpu.VMEM((CH2,), jnp.int32),
            pltpu.VMEM((CH2,), jnp.int32),
            pltpu.VMEM((CH2, 4), jnp.float32),
            pltpu.VMEM((CH2, 4), jnp.float32),
            pltpu.VMEM((CH2, 64), jnp.float32),
            pltpu.VMEM((CH2, 64), jnp.float32),
            pltpu.VMEM((CH2, MSGW), jnp.float32),
            pltpu.VMEM_SHARED((N, MSGW), jnp.float32),
            pltpu.SemaphoreType.DMA,
            pltpu.SemaphoreType.DMA,
            pltpu.SemaphoreType.DMA,
            pltpu.SemaphoreType.DMA,
        ],
    )
    return f(edge_index, xlo, xhi, w4, zrows)


def _post_body(a_r, d_r, x_r, mask_r, Wf_r, csum_r, W1_r, b1_r,
               W2_r, b2_r, gid_r, samp_r, logp_r):
    den3 = d_r[0, :, :] + d_r[1, :, :]
    has = den3[:, 0:1] > 0
    osum = jnp.where(has, csum_r[...], 0.0)
    for h in range(HEADS):
        den = den3[:, h:h + 1]
        agg = jnp.concatenate(
            [a_r[0, :, h * 64:(h + 1) * 64], a_r[1, :, h * 64:(h + 1) * 64]],
            axis=1)
        agg = jnp.where(den > 0, agg / jnp.where(den > 0, den, 1.0), 0.0)
        osum = osum + jnp.dot(agg, Wf_r[h, :, :],
                              preferred_element_type=jnp.float32)
    enc = osum * (1.0 / 3.0)
    enc = jnp.where(enc > 0, enc, jnp.exp(jnp.minimum(enc, 0.0)) - 1.0)
    encg = enc.reshape(NB // CITIES, CITIES, 2 * H)
    center = jnp.broadcast_to(encg[:, 0:1, :], encg.shape)
    crep = center.reshape(NB, 2 * H)
    h1 = (jnp.einsum('nc,kc->nk', crep, W1_r[:, :2 * H],
                     preferred_element_type=jnp.float32)
          + jnp.einsum('nc,kc->nk', enc, W1_r[:, 2 * H:],
                       preferred_element_type=jnp.float32)
          + b1_r[...])
    h1 = jnp.maximum(h1, 0.0)
    lg = (jnp.einsum('nf,of->no', x_r[...], W2_r[:, :F],
                     preferred_element_type=jnp.float32)
          + jnp.einsum('nh,oh->no', h1, W2_r[:, F:],
                       preferred_element_type=jnp.float32)
          + b2_r[...])
    lg = lg - jnp.where(mask_r[...], 0.0, 1e6)
    iota = lax.broadcasted_iota(jnp.int32, (CITIES, 1), 0)
    for g in range(NB // CITIES):
        l = lg[g * CITIES:(g + 1) * CITIES, :]
        m = jnp.max(l, axis=0, keepdims=True)
        den = jnp.sum(jnp.exp(l - m), axis=0, keepdims=True)
        cand = jnp.where(l >= m, iota, N)
        samp = jnp.min(cand, axis=0, keepdims=True)
        samp_r[0, pl.ds(g, 1), :] = samp + gid_r[0, pl.ds(g, 1), :]
        logp_r[0, pl.ds(g, 1), :] = -jnp.log(den)


def _post(acc, den, x, mask, Wf, csum, W1, b1, W2, b2, gid2d):
    return pl.pallas_call(
        _post_body,
        grid=(NBLK,),
        in_specs=[
            pl.BlockSpec((2, NB, MSGW), lambda i: (0, i, 0)),
            pl.BlockSpec((2, NB, 16), lambda i: (0, i, 0)),
            pl.BlockSpec((NB, F), lambda i: (i, 0)),
            pl.BlockSpec((NB, 1), lambda i: (i, 0)),
            pl.BlockSpec((HEADS, F, 2 * H), lambda i: (0, 0, 0)),
            pl.BlockSpec((1, 2 * H), lambda i: (0, 0)),
            pl.BlockSpec((H, 4 * H), lambda i: (0, 0)),
            pl.BlockSpec((1, H), lambda i: (0, 0)),
            pl.BlockSpec((1, H + F), lambda i: (0, 0)),
            pl.BlockSpec((1, 1), lambda i: (0, 0)),
            pl.BlockSpec((1, G // NBLK, 1), lambda i: (i, 0, 0)),
        ],
        out_specs=(
            pl.BlockSpec((1, G // NBLK, 1), lambda i: (i, 0, 0)),
            pl.BlockSpec((1, G // NBLK, 1), lambda i: (i, 0, 0)),
        ),
        out_shape=(
            jax.ShapeDtypeStruct((NBLK, G // NBLK, 1), jnp.int32),
            jax.ShapeDtypeStruct((NBLK, G // NBLK, 1), jnp.float32),
        ),
    )(acc, den, x, mask, Wf, csum, W1, b1.reshape(1, H), W2,
      b2.reshape(1, 1), gid2d)


def kernel(x, edge_index, batch, center_node_index, mask, graph_id_index,
           W0, b0, initial_embed, Wg, a_src, a_dst, W1, b1, W2, b2):
    Wf, csum, T, xlo, xhi, M = _prep(x, W0, b0, initial_embed, Wg,
                                     a_src, a_dst)
    zden = jnp.zeros((ROWS_PER_TILE, 16), jnp.float32)
    zmsg = jnp.zeros((ROWS_PER_TILE, MSGW), jnp.float32)
    den, w4 = _sc_denominators(edge_index, T, M, zden)
    acc = _sc_messages(edge_index, xlo, xhi, w4, zmsg)
    samp, logp = _post(acc, den, x, mask, Wf, csum,
                       W1, b1, W2, b2,
                       graph_id_index.reshape(NBLK, G // NBLK, 1))
    return samp.reshape(G), logp.reshape(G)
